# Initial kernel scaffold; baseline (speedup 1.0000x reference)
import jax
import jax.numpy as jnp
from jax.experimental import pallas as pl


def kernel(x, edge_index, sage_Wl, sage_Wr, sage_b, gat_W, gat_asrc, gat_adst, gat_b, pW1, pb1, pW2, pb2, pW3, pb3):
    raise NotImplementedError("write your pallas kernel here")


# speedup vs baseline: 6.6203x; 6.6203x over previous
import functools

import jax
import jax.numpy as jnp
from jax import lax
from jax.experimental import pallas as pl
from jax.experimental.pallas import tpu as pltpu
from jax.experimental.pallas import tpu_sc as plsc

N = 50000
D = 64
E = 800000
NC = 2
NS = 16
LANES = 16
EPAD = 851968
EROWS = EPAD // 128
NACC = 51200
RPT = NACC // NS
CHUNK = 2048
ECH = 512
IR = ECH // 128
ROWS_T = EROWS // NS
ROWS_W = EROWS // (NS * NC)

@functools.lru_cache(maxsize=None)
def _mesh():
  return plsc.VectorSubcoreMesh(
      core_axis_name="c", subcore_axis_name="s", num_cores=NC,
      num_subcores=NS)


def _zero_flat(buf, nwords):
  z = jnp.zeros((16,), jnp.float32)
  def body(k, _):
    buf[pl.ds(k * 16, 16)] = z
    return 0
  lax.fori_loop(0, nwords // 16, body, 0)


def _zero_rows(buf, nrows):
  z = jnp.zeros((16,), jnp.float32)
  def body(e, _):
    buf[e, pl.ds(0, 16)] = z
    buf[e, pl.ds(16, 16)] = z
    return 0
  lax.fori_loop(0, nrows, body, 0)


def _count_body(dst_r, out, dbuf, ones, zflat, acc):
  cid = lax.axis_index("c")
  sid = lax.axis_index("s")
  wid = sid * NC + cid
  one = jnp.full((16,), 1.0, jnp.float32)
  for k in range(8):
    ones[0, pl.ds(k * 16, 16)] = one
  _zero_flat(zflat, CHUNK)
  base = sid * RPT
  pltpu.sync_copy(zflat, acc.at[pl.ds(base, CHUNK)])
  pltpu.sync_copy(zflat.at[pl.ds(0, RPT - CHUNK)],
                  acc.at[pl.ds(base + CHUNK, RPT - CHUNK)])
  plsc.subcore_barrier()
  def chunk(t, _):
    rowbase = wid * ROWS_W + t * 16
    pltpu.sync_copy(dst_r.at[pl.ds(rowbase, 16)], dbuf)
    for j in range(16):
      pltpu.sync_copy(ones.at[0], acc.at[dbuf.at[j]], add=True)
    return 0
  lax.fori_loop(0, ROWS_W // 16, chunk, 0)
  plsc.subcore_barrier()
  pltpu.sync_copy(acc.at[pl.ds(base, RPT)], out.at[cid, pl.ds(base, RPT)])


@functools.lru_cache(maxsize=None)
def _sc_count():
  return pl.kernel(
    _count_body,
    out_type=jax.ShapeDtypeStruct((NC, NACC), jnp.float32),
    mesh=_mesh(),
    compiler_params=pltpu.CompilerParams(use_tc_tiling_on_sc=False, needs_layout_passes=False),
    scratch_types=[
        pltpu.VMEM((16, 128), jnp.int32),
        pltpu.VMEM((1, 128), jnp.float32),
        pltpu.VMEM((CHUNK,), jnp.float32),
        pltpu.VMEM_SHARED((NACC,), jnp.float32),
    ],
  )


def _rows_body(h0, h1, src_r, dst_r, wt_r, out, sbuf, dbuf, wbuf, rows, acc,
               sem, *, weighted):
  cid = lax.axis_index("c")
  sid = lax.axis_index("s")
  _zero_rows(rows, ECH)
  base = sid * RPT
  for k in range(RPT // ECH):
    pltpu.sync_copy(rows, acc.at[pl.ds(base + k * ECH, ECH)])
  rem = RPT % ECH
  if rem:
    pltpu.sync_copy(rows.at[pl.ds(0, rem)],
                    acc.at[pl.ds(base + (RPT // ECH) * ECH, rem)])
  plsc.subcore_barrier()

  def main(table):
    def chunk(t, _):
      rowbase = sid * ROWS_T + t * IR
      pltpu.sync_copy(src_r.at[pl.ds(rowbase, IR)], sbuf)
      pltpu.sync_copy(dst_r.at[pl.ds(rowbase, IR)], dbuf)
      if weighted:
        pltpu.sync_copy(wt_r.at[pl.ds(rowbase, IR)], wbuf)
      cps = [pltpu.async_copy(table.at[sbuf.at[j]],
                              rows.at[pl.ds(j * 128, 128)], sem)
             for j in range(IR)]
      for cp in cps:
        cp.wait()
      if weighted:
        lanes = jax.lax.broadcasted_iota(jnp.int32, (16,), 0)
        for j in range(IR):
          def scale(k, _):
            w16 = wbuf[j, pl.ds(k * 16, 16)]
            e16 = (j * 128 + k * 16) + lanes
            for c in range(32):
              cv = jnp.full((16,), c, jnp.int32)
              v = plsc.load_gather(rows, [e16, cv])
              plsc.store_scatter(rows, [e16, cv], v * w16)
            return 0
          lax.fori_loop(0, 8, scale, 0)
      for j in range(IR):
        pltpu.sync_copy(rows.at[pl.ds(j * 128, 128)],
                        acc.at[dbuf.at[j]], add=True)
      return 0
    lax.fori_loop(0, ROWS_T // IR, chunk, 0)

  @pl.when(cid == 0)
  def _():
    main(h0)

  @pl.when(cid == 1)
  def _():
    main(h1)

  plsc.subcore_barrier()
  pltpu.sync_copy(acc.at[pl.ds(base, RPT)], out.at[cid, pl.ds(base, RPT)])


@functools.lru_cache(maxsize=None)
def _make_rows_kernel(weighted):
  body = functools.partial(_rows_body, weighted=weighted)
  return pl.kernel(
      body,
      out_type=jax.ShapeDtypeStruct((NC, NACC, 32), jnp.float32),
      mesh=_mesh(),
      compiler_params=pltpu.CompilerParams(use_tc_tiling_on_sc=False, needs_layout_passes=False),
      scratch_types=[
          pltpu.VMEM((IR, 128), jnp.int32),
          pltpu.VMEM((IR, 128), jnp.int32),
          pltpu.VMEM((IR, 128), jnp.float32),
          pltpu.VMEM((ECH, 32), jnp.float32),
          pltpu.VMEM_SHARED((NACC, 32), jnp.float32),
          pltpu.SemaphoreType.DMA,
      ],
  )


def _sc_rows(*a):
  return _make_rows_kernel(False)(*a)


def _sc_rows_w(*a):
  return _make_rows_kernel(True)(*a)


def _scalar_body(asc_h, adst_h, src_r, dst_r, gmax_h, ex_out, den_out,
                 asct, adstt, sbuf, dbuf, exbuf, gv, zflat, acc):
  cid = lax.axis_index("c")
  sid = lax.axis_index("s")
  wid = sid * NC + cid
  pltpu.sync_copy(asc_h, asct)
  pltpu.sync_copy(adst_h, adstt)
  pltpu.sync_copy(gmax_h, gv)
  _zero_flat(zflat, CHUNK)
  base = sid * RPT
  pltpu.sync_copy(zflat, acc.at[pl.ds(base, CHUNK)])
  pltpu.sync_copy(zflat.at[pl.ds(0, RPT - CHUNK)],
                  acc.at[pl.ds(base + CHUNK, RPT - CHUNK)])
  plsc.subcore_barrier()
  g = gv[...]
  nmax = jnp.full((16,), N - 1, jnp.int32)
  def chunk(t, _):
    rowbase = wid * ROWS_W + t * 16
    pltpu.sync_copy(src_r.at[pl.ds(rowbase, 16)], sbuf)
    pltpu.sync_copy(dst_r.at[pl.ds(rowbase, 16)], dbuf)
    for j in range(16):
      def grp(k, _):
        s16 = sbuf[j, pl.ds(k * 16, 16)]
        d16 = jnp.minimum(dbuf[j, pl.ds(k * 16, 16)], nmax)
        a_s = plsc.load_gather(asct, [s16])
        a_d = plsc.load_gather(adstt, [d16])
        t1 = a_s + a_d
        lt = jnp.maximum(t1, 0.2 * t1)
        u = g + a_d
        lu = jnp.maximum(u, 0.2 * u)
        exbuf[j, pl.ds(k * 16, 16)] = jnp.exp(lt - lu)
        return 0
      lax.fori_loop(0, 8, grp, 0)
    pltpu.sync_copy(exbuf, ex_out.at[pl.ds(rowbase, 16)])
    for j in range(16):
      pltpu.sync_copy(exbuf.at[j], acc.at[dbuf.at[j]], add=True)
    return 0
  lax.fori_loop(0, ROWS_W // 16, chunk, 0)
  plsc.subcore_barrier()
  pltpu.sync_copy(acc.at[pl.ds(base, RPT)], den_out.at[cid, pl.ds(base, RPT)])


@functools.lru_cache(maxsize=None)
def _sc_scalar():
  return pl.kernel(
    _scalar_body,
    out_type=(jax.ShapeDtypeStruct((EROWS, 128), jnp.float32),
              jax.ShapeDtypeStruct((NC, NACC), jnp.float32)),
    mesh=_mesh(),
    compiler_params=pltpu.CompilerParams(use_tc_tiling_on_sc=False, needs_layout_passes=False),
    scratch_types=[
        pltpu.VMEM((NACC,), jnp.float32),
        pltpu.VMEM((NACC,), jnp.float32),
        pltpu.VMEM((16, 128), jnp.int32),
        pltpu.VMEM((16, 128), jnp.int32),
        pltpu.VMEM((16, 128), jnp.float32),
        pltpu.VMEM((16,), jnp.float32),
        pltpu.VMEM((CHUNK,), jnp.float32),
        pltpu.VMEM_SHARED((NACC,), jnp.float32),
    ],
  )


BR = 2048
GRID = NACC // BR


def _mm_t(a, w):
  return lax.dot_general(a, w, (((1,), (1,)), ((), ())),
                         preferred_element_type=jnp.float32)


def _sage_tc_body(s2, cnt2, h, wl, wr, b, o, oh, *, do_relu):
  s = jnp.concatenate([s2[0], s2[1]], axis=-1)
  cnt = jnp.clip(cnt2[0] + cnt2[1], 1.0, None)[:, None]
  res = _mm_t(s / cnt, wl[...]) + _mm_t(h[...], wr[...]) + b[...]
  if do_relu:
    res = jnp.maximum(res, 0.0)
  o[...] = res
  oh[0] = res[:, :32]
  oh[1] = res[:, 32:]


def _sage_tc(s2, cnt2, h, wl, wr, b, do_relu):
  body = functools.partial(_sage_tc_body, do_relu=do_relu)
  return pl.pallas_call(
      body,
      grid=(GRID,),
      in_specs=[
          pl.BlockSpec((NC, BR, 32), lambda i: (0, i, 0)),
          pl.BlockSpec((NC, BR), lambda i: (0, i)),
          pl.BlockSpec((BR, D), lambda i: (i, 0)),
          pl.BlockSpec((D, D), lambda i: (0, 0)),
          pl.BlockSpec((D, D), lambda i: (0, 0)),
          pl.BlockSpec((1, D), lambda i: (0, 0)),
      ],
      out_specs=[
          pl.BlockSpec((BR, D), lambda i: (i, 0)),
          pl.BlockSpec((NC, BR, 32), lambda i: (0, i, 0)),
      ],
      out_shape=[
          jax.ShapeDtypeStruct((NACC, D), jnp.float32),
          jax.ShapeDtypeStruct((NC, NACC, 32), jnp.float32),
      ],
  )(s2, cnt2, h, wl, wr, b)


def _gat_pre_body(h, w, a_s, a_d, ohw, oasc, oadst, ogmax):
  i = pl.program_id(0)
  hw = _mm_t(h[...], w[...])
  asc = _mm_t(hw, a_s[...])
  adst = _mm_t(hw, a_d[...])
  ohw[0] = hw[:, :32]
  ohw[1] = hw[:, 32:]
  oasc[...] = asc
  oadst[...] = adst
  bmax = jnp.max(asc)

  @pl.when(i == 0)
  def _():
    ogmax[0, 0] = bmax

  @pl.when(i > 0)
  def _():
    ogmax[0, 0] = jnp.maximum(ogmax[0, 0], bmax)


def _gat_pre(h, w, a_s2, a_d2):
  return pl.pallas_call(
      _gat_pre_body,
      grid=(GRID,),
      in_specs=[
          pl.BlockSpec((BR, D), lambda i: (i, 0)),
          pl.BlockSpec((D, D), lambda i: (0, 0)),
          pl.BlockSpec((1, D), lambda i: (0, 0)),
          pl.BlockSpec((1, D), lambda i: (0, 0)),
      ],
      out_specs=[
          pl.BlockSpec((NC, BR, 32), lambda i: (0, i, 0)),
          pl.BlockSpec((BR, 1), lambda i: (i, 0)),
          pl.BlockSpec((BR, 1), lambda i: (i, 0)),
          pl.BlockSpec((1, 1), lambda i: (0, 0), memory_space=pltpu.SMEM),
      ],
      out_shape=[
          jax.ShapeDtypeStruct((NC, NACC, 32), jnp.float32),
          jax.ShapeDtypeStruct((NACC, 1), jnp.float32),
          jax.ShapeDtypeStruct((NACC, 1), jnp.float32),
          jax.ShapeDtypeStruct((1, 1), jnp.float32),
      ],
  )(h, w, a_s2, a_d2)


def _gat_post_body(n2, d2, asc, adst, gmax, hw2, b, o, oh, *, do_relu):
  num = jnp.concatenate([n2[0], n2[1]], axis=-1)
  den = (d2[0] + d2[1])[:, None]
  hw = jnp.concatenate([hw2[0], hw2[1]], axis=-1)
  g = gmax[0, 0]
  t1 = asc[...] + adst[...]
  lt = jnp.maximum(t1, 0.2 * t1)
  u = g + adst[...]
  lu = jnp.maximum(u, 0.2 * u)
  exs = jnp.exp(lt - lu)
  res = (num + exs * hw) / (den + exs) + b[...]
  if do_relu:
    res = jnp.maximum(res, 0.0)
  o[...] = res
  oh[0] = res[:, :32]
  oh[1] = res[:, 32:]


def _gat_post(n2, d2, asc, adst, gmax, hw2, b, do_relu):
  body = functools.partial(_gat_post_body, do_relu=do_relu)
  return pl.pallas_call(
      body,
      grid=(GRID,),
      in_specs=[
          pl.BlockSpec((NC, BR, 32), lambda i: (0, i, 0)),
          pl.BlockSpec((NC, BR), lambda i: (0, i)),
          pl.BlockSpec((BR, 1), lambda i: (i, 0)),
          pl.BlockSpec((BR, 1), lambda i: (i, 0)),
          pl.BlockSpec((1, 1), lambda i: (0, 0), memory_space=pltpu.SMEM),
          pl.BlockSpec((NC, BR, 32), lambda i: (0, i, 0)),
          pl.BlockSpec((1, D), lambda i: (0, 0)),
      ],
      out_specs=[
          pl.BlockSpec((BR, D), lambda i: (i, 0)),
          pl.BlockSpec((NC, BR, 32), lambda i: (0, i, 0)),
      ],
      out_shape=[
          jax.ShapeDtypeStruct((NACC, D), jnp.float32),
          jax.ShapeDtypeStruct((NC, NACC, 32), jnp.float32),
      ],
  )(n2, d2, asc, adst, gmax, hw2, b)


def _proj_body(h, w1, b1, w2, b2, w3, b3, o):
  r = jnp.maximum(_mm_t(h[...], w1[...]) + b1[...], 0.0)
  r = jnp.maximum(_mm_t(r, w2[...]) + b2[...], 0.0)
  o[...] = _mm_t(r, w3[...]) + b3[...]


def _proj(h, w1, b1, w2, b2, w3, b3):
  return pl.pallas_call(
      _proj_body,
      grid=(GRID,),
      in_specs=[
          pl.BlockSpec((BR, D), lambda i: (i, 0)),
          pl.BlockSpec((64, 64), lambda i: (0, 0)),
          pl.BlockSpec((1, 64), lambda i: (0, 0)),
          pl.BlockSpec((32, 64), lambda i: (0, 0)),
          pl.BlockSpec((1, 32), lambda i: (0, 0)),
          pl.BlockSpec((16, 32), lambda i: (0, 0)),
          pl.BlockSpec((1, 16), lambda i: (0, 0)),
      ],
      out_specs=pl.BlockSpec((BR, 16), lambda i: (i, 0)),
      out_shape=jax.ShapeDtypeStruct((NACC, 16), jnp.float32),
  )(h, w1, b1, w2, b2, w3, b3)


@jax.jit
def kernel(x, edge_index, sage_Wl, sage_Wr, sage_b, gat_W, gat_asrc, gat_adst,
           gat_b, pW1, pb1, pW2, pb2, pW3, pb3):
  src = edge_index[0]
  dst = edge_index[1]
  npad = EPAD - E
  src_r = jnp.concatenate([src, jnp.zeros((npad,), jnp.int32)]).reshape(
      EROWS, 128)
  dst_r = jnp.concatenate([dst, jnp.full((npad,), N, jnp.int32)]).reshape(
      EROWS, 128)

  cnt2 = _sc_count()(dst_r)

  xp = jnp.pad(x, ((0, NACC - N), (0, 0)))
  h = xp
  hh = jnp.stack([xp[:, :32], xp[:, 32:]])
  for i in range(4):
    s2 = _sc_rows(hh[0], hh[1], src_r, dst_r, jnp.zeros((EROWS, 128),
                                                        jnp.float32))
    h, hh = _sage_tc(s2, cnt2, h, sage_Wl[i], sage_Wr[i],
                     sage_b[i].reshape(1, D), i < 3)

  for i in range(3):
    hw2, asc, adst, gmax = _gat_pre(h, gat_W[i], gat_asrc[i].reshape(1, D),
                                    gat_adst[i].reshape(1, D))
    gmax16 = jnp.full((16,), 1.0, jnp.float32) * gmax[0, 0]
    ex_r, den2 = _sc_scalar()(asc.reshape(NACC), adst.reshape(NACC), src_r, dst_r,
                              gmax16)
    n2 = _sc_rows_w(hw2[0], hw2[1], src_r, dst_r, ex_r)
    h, hh = _gat_post(n2, den2, asc, adst, gmax, hw2,
                      gat_b[i].reshape(1, D), i < 2)

  out = _proj(h, pW1, pb1.reshape(1, 64), pW2, pb2.reshape(1, 32),
              pW3, pb3.reshape(1, 16))
  return out[:N]


# speedup vs baseline: 10.6568x; 1.6097x over previous
import functools

import jax
import jax.numpy as jnp
from jax import lax
from jax.experimental import pallas as pl
from jax.experimental.pallas import tpu as pltpu
from jax.experimental.pallas import tpu_sc as plsc

N = 50000
D = 64
E = 800000
NC = 2
NS = 16
LANES = 16
EPAD = 851968
EROWS = EPAD // 128
NACC = 51200
RPT = NACC // NS
CHUNK = 2048
ECH = 512
IR = ECH // 128
ROWS_T = EROWS // NS
ROWS_W = EROWS // (NS * NC)

@functools.lru_cache(maxsize=None)
def _mesh():
  return plsc.VectorSubcoreMesh(
      core_axis_name="c", subcore_axis_name="s", num_cores=NC,
      num_subcores=NS)


def _zero_flat(buf, nwords):
  z = jnp.zeros((16,), jnp.float32)
  def body(k, _):
    buf[pl.ds(k * 16, 16)] = z
    return 0
  lax.fori_loop(0, nwords // 16, body, 0)


def _zero_rows(buf, nrows):
  z = jnp.zeros((16,), jnp.float32)
  def body(e, _):
    buf[e, pl.ds(0, 16)] = z
    buf[e, pl.ds(16, 16)] = z
    return 0
  lax.fori_loop(0, nrows, body, 0)


def _count_body(dst_r, out, dbuf, ones, zflat, acc):
  cid = lax.axis_index("c")
  sid = lax.axis_index("s")
  wid = sid * NC + cid
  one = jnp.full((16,), 1.0, jnp.float32)
  for k in range(8):
    ones[0, pl.ds(k * 16, 16)] = one
  _zero_flat(zflat, CHUNK)
  base = sid * RPT
  pltpu.sync_copy(zflat, acc.at[pl.ds(base, CHUNK)])
  pltpu.sync_copy(zflat.at[pl.ds(0, RPT - CHUNK)],
                  acc.at[pl.ds(base + CHUNK, RPT - CHUNK)])
  plsc.subcore_barrier()
  def chunk(t, _):
    rowbase = wid * ROWS_W + t * 16
    pltpu.sync_copy(dst_r.at[pl.ds(rowbase, 16)], dbuf)
    for j in range(16):
      pltpu.sync_copy(ones.at[0], acc.at[dbuf.at[j]], add=True)
    return 0
  lax.fori_loop(0, ROWS_W // 16, chunk, 0)
  plsc.subcore_barrier()
  pltpu.sync_copy(acc.at[pl.ds(base, RPT)], out.at[cid, pl.ds(base, RPT)])


@functools.lru_cache(maxsize=None)
def _sc_count():
  return pl.kernel(
    _count_body,
    out_type=jax.ShapeDtypeStruct((NC, NACC), jnp.float32),
    mesh=_mesh(),
    compiler_params=pltpu.CompilerParams(use_tc_tiling_on_sc=False, needs_layout_passes=False),
    scratch_types=[
        pltpu.VMEM((16, 128), jnp.int32),
        pltpu.VMEM((1, 128), jnp.float32),
        pltpu.VMEM((CHUNK,), jnp.float32),
        pltpu.VMEM_SHARED((NACC,), jnp.float32),
    ],
  )


def _rows_body(h0, h1, src_r, dst_r, wt_r, out, sbuf, dbuf, wbuf, rows, acc,
               sem, *, weighted):
  cid = lax.axis_index("c")
  sid = lax.axis_index("s")
  _zero_rows(rows, ECH)
  base = sid * RPT
  for k in range(RPT // ECH):
    pltpu.sync_copy(rows, acc.at[pl.ds(base + k * ECH, ECH)])
  rem = RPT % ECH
  if rem:
    pltpu.sync_copy(rows.at[pl.ds(0, rem)],
                    acc.at[pl.ds(base + (RPT // ECH) * ECH, rem)])
  plsc.subcore_barrier()

  def main(table):
    def chunk(t, _):
      rowbase = sid * ROWS_T + t * IR
      pltpu.sync_copy(src_r.at[pl.ds(rowbase, IR)], sbuf)
      pltpu.sync_copy(dst_r.at[pl.ds(rowbase, IR)], dbuf)
      if weighted:
        pltpu.sync_copy(wt_r.at[pl.ds(rowbase, IR)], wbuf)
      cps = [pltpu.async_copy(table.at[sbuf.at[j]],
                              rows.at[pl.ds(j * 128, 128)], sem)
             for j in range(IR)]
      for cp in cps:
        cp.wait()
      if weighted:
        for j in range(IR):
          def scale(k, _):
            w16 = wbuf[j, pl.ds(k * 16, 16)]
            for l in range(16):
              e = j * 128 + k * 16 + l
              wb = jnp.take_along_axis(w16, jnp.full((16,), l, jnp.int32),
                                       axis=0)
              rows[e, pl.ds(0, 16)] = rows[e, pl.ds(0, 16)] * wb
              rows[e, pl.ds(16, 16)] = rows[e, pl.ds(16, 16)] * wb
            return 0
          lax.fori_loop(0, 8, scale, 0)
      for j in range(IR):
        pltpu.sync_copy(rows.at[pl.ds(j * 128, 128)],
                        acc.at[dbuf.at[j]], add=True)
      return 0
    lax.fori_loop(0, ROWS_T // IR, chunk, 0)

  @pl.when(cid == 0)
  def _():
    main(h0)

  @pl.when(cid == 1)
  def _():
    main(h1)

  plsc.subcore_barrier()
  pltpu.sync_copy(acc.at[pl.ds(base, RPT)], out.at[cid, pl.ds(base, RPT)])


@functools.lru_cache(maxsize=None)
def _make_rows_kernel(weighted):
  body = functools.partial(_rows_body, weighted=weighted)
  return pl.kernel(
      body,
      out_type=jax.ShapeDtypeStruct((NC, NACC, 32), jnp.float32),
      mesh=_mesh(),
      compiler_params=pltpu.CompilerParams(use_tc_tiling_on_sc=False, needs_layout_passes=False),
      scratch_types=[
          pltpu.VMEM((IR, 128), jnp.int32),
          pltpu.VMEM((IR, 128), jnp.int32),
          pltpu.VMEM((IR, 128), jnp.float32),
          pltpu.VMEM((ECH, 32), jnp.float32),
          pltpu.VMEM_SHARED((NACC, 32), jnp.float32),
          pltpu.SemaphoreType.DMA,
      ],
  )


def _sc_rows(*a):
  return _make_rows_kernel(False)(*a)


def _sc_rows_w(*a):
  return _make_rows_kernel(True)(*a)


def _scalar_body(asc_h, adst_h, src_r, dst_r, gmax_h, ex_out, den_out,
                 asct, adstt, sbuf, dbuf, exbuf, gv, zflat, acc):
  cid = lax.axis_index("c")
  sid = lax.axis_index("s")
  wid = sid * NC + cid
  pltpu.sync_copy(asc_h, asct)
  pltpu.sync_copy(adst_h, adstt)
  pltpu.sync_copy(gmax_h, gv)
  _zero_flat(zflat, CHUNK)
  base = sid * RPT
  pltpu.sync_copy(zflat, acc.at[pl.ds(base, CHUNK)])
  pltpu.sync_copy(zflat.at[pl.ds(0, RPT - CHUNK)],
                  acc.at[pl.ds(base + CHUNK, RPT - CHUNK)])
  plsc.subcore_barrier()
  g = gv[...]
  nmax = jnp.full((16,), N - 1, jnp.int32)
  def chunk(t, _):
    rowbase = wid * ROWS_W + t * 16
    pltpu.sync_copy(src_r.at[pl.ds(rowbase, 16)], sbuf)
    pltpu.sync_copy(dst_r.at[pl.ds(rowbase, 16)], dbuf)
    for j in range(16):
      def grp(k, _):
        s16 = sbuf[j, pl.ds(k * 16, 16)]
        d16 = jnp.minimum(dbuf[j, pl.ds(k * 16, 16)], nmax)
        a_s = plsc.load_gather(asct, [s16])
        a_d = plsc.load_gather(adstt, [d16])
        t1 = a_s + a_d
        lt = jnp.maximum(t1, 0.2 * t1)
        u = g + a_d
        lu = jnp.maximum(u, 0.2 * u)
        exbuf[j, pl.ds(k * 16, 16)] = jnp.exp(lt - lu)
        return 0
      lax.fori_loop(0, 8, grp, 0)
    pltpu.sync_copy(exbuf, ex_out.at[pl.ds(rowbase, 16)])
    for j in range(16):
      pltpu.sync_copy(exbuf.at[j], acc.at[dbuf.at[j]], add=True)
    return 0
  lax.fori_loop(0, ROWS_W // 16, chunk, 0)
  plsc.subcore_barrier()
  pltpu.sync_copy(acc.at[pl.ds(base, RPT)], den_out.at[cid, pl.ds(base, RPT)])


@functools.lru_cache(maxsize=None)
def _sc_scalar():
  return pl.kernel(
    _scalar_body,
    out_type=(jax.ShapeDtypeStruct((EROWS, 128), jnp.float32),
              jax.ShapeDtypeStruct((NC, NACC), jnp.float32)),
    mesh=_mesh(),
    compiler_params=pltpu.CompilerParams(use_tc_tiling_on_sc=False, needs_layout_passes=False),
    scratch_types=[
        pltpu.VMEM((NACC,), jnp.float32),
        pltpu.VMEM((NACC,), jnp.float32),
        pltpu.VMEM((16, 128), jnp.int32),
        pltpu.VMEM((16, 128), jnp.int32),
        pltpu.VMEM((16, 128), jnp.float32),
        pltpu.VMEM((16,), jnp.float32),
        pltpu.VMEM((CHUNK,), jnp.float32),
        pltpu.VMEM_SHARED((NACC,), jnp.float32),
    ],
  )


BR = 2048
GRID = NACC // BR


def _mm_t(a, w):
  return lax.dot_general(a, w, (((1,), (1,)), ((), ())),
                         preferred_element_type=jnp.float32)


def _sage_tc_body(s2, cnt2, h, wl, wr, b, o, oh, *, do_relu):
  s = jnp.concatenate([s2[0], s2[1]], axis=-1)
  cnt = jnp.clip(cnt2[0] + cnt2[1], 1.0, None)[:, None]
  res = _mm_t(s / cnt, wl[...]) + _mm_t(h[...], wr[...]) + b[...]
  if do_relu:
    res = jnp.maximum(res, 0.0)
  o[...] = res
  oh[0] = res[:, :32]
  oh[1] = res[:, 32:]


def _sage_tc(s2, cnt2, h, wl, wr, b, do_relu):
  body = functools.partial(_sage_tc_body, do_relu=do_relu)
  return pl.pallas_call(
      body,
      grid=(GRID,),
      in_specs=[
          pl.BlockSpec((NC, BR, 32), lambda i: (0, i, 0)),
          pl.BlockSpec((NC, BR), lambda i: (0, i)),
          pl.BlockSpec((BR, D), lambda i: (i, 0)),
          pl.BlockSpec((D, D), lambda i: (0, 0)),
          pl.BlockSpec((D, D), lambda i: (0, 0)),
          pl.BlockSpec((1, D), lambda i: (0, 0)),
      ],
      out_specs=[
          pl.BlockSpec((BR, D), lambda i: (i, 0)),
          pl.BlockSpec((NC, BR, 32), lambda i: (0, i, 0)),
      ],
      out_shape=[
          jax.ShapeDtypeStruct((NACC, D), jnp.float32),
          jax.ShapeDtypeStruct((NC, NACC, 32), jnp.float32),
      ],
  )(s2, cnt2, h, wl, wr, b)


def _gat_pre_body(h, w, a_s, a_d, ohw, oasc, oadst, ogmax):
  i = pl.program_id(0)
  hw = _mm_t(h[...], w[...])
  asc = _mm_t(hw, a_s[...])
  adst = _mm_t(hw, a_d[...])
  ohw[0] = hw[:, :32]
  ohw[1] = hw[:, 32:]
  oasc[...] = asc
  oadst[...] = adst
  bmax = jnp.max(asc)

  @pl.when(i == 0)
  def _():
    ogmax[0, 0] = bmax

  @pl.when(i > 0)
  def _():
    ogmax[0, 0] = jnp.maximum(ogmax[0, 0], bmax)


def _gat_pre(h, w, a_s2, a_d2):
  return pl.pallas_call(
      _gat_pre_body,
      grid=(GRID,),
      in_specs=[
          pl.BlockSpec((BR, D), lambda i: (i, 0)),
          pl.BlockSpec((D, D), lambda i: (0, 0)),
          pl.BlockSpec((1, D), lambda i: (0, 0)),
          pl.BlockSpec((1, D), lambda i: (0, 0)),
      ],
      out_specs=[
          pl.BlockSpec((NC, BR, 32), lambda i: (0, i, 0)),
          pl.BlockSpec((BR, 1), lambda i: (i, 0)),
          pl.BlockSpec((BR, 1), lambda i: (i, 0)),
          pl.BlockSpec((1, 1), lambda i: (0, 0), memory_space=pltpu.SMEM),
      ],
      out_shape=[
          jax.ShapeDtypeStruct((NC, NACC, 32), jnp.float32),
          jax.ShapeDtypeStruct((NACC, 1), jnp.float32),
          jax.ShapeDtypeStruct((NACC, 1), jnp.float32),
          jax.ShapeDtypeStruct((1, 1), jnp.float32),
      ],
  )(h, w, a_s2, a_d2)


def _gat_post_body(n2, d2, asc, adst, gmax, hw2, b, o, oh, *, do_relu):
  num = jnp.concatenate([n2[0], n2[1]], axis=-1)
  den = (d2[0] + d2[1])[:, None]
  hw = jnp.concatenate([hw2[0], hw2[1]], axis=-1)
  g = gmax[0, 0]
  t1 = asc[...] + adst[...]
  lt = jnp.maximum(t1, 0.2 * t1)
  u = g + adst[...]
  lu = jnp.maximum(u, 0.2 * u)
  exs = jnp.exp(lt - lu)
  res = (num + exs * hw) / (den + exs) + b[...]
  if do_relu:
    res = jnp.maximum(res, 0.0)
  o[...] = res
  oh[0] = res[:, :32]
  oh[1] = res[:, 32:]


def _gat_post(n2, d2, asc, adst, gmax, hw2, b, do_relu):
  body = functools.partial(_gat_post_body, do_relu=do_relu)
  return pl.pallas_call(
      body,
      grid=(GRID,),
      in_specs=[
          pl.BlockSpec((NC, BR, 32), lambda i: (0, i, 0)),
          pl.BlockSpec((NC, BR), lambda i: (0, i)),
          pl.BlockSpec((BR, 1), lambda i: (i, 0)),
          pl.BlockSpec((BR, 1), lambda i: (i, 0)),
          pl.BlockSpec((1, 1), lambda i: (0, 0), memory_space=pltpu.SMEM),
          pl.BlockSpec((NC, BR, 32), lambda i: (0, i, 0)),
          pl.BlockSpec((1, D), lambda i: (0, 0)),
      ],
      out_specs=[
          pl.BlockSpec((BR, D), lambda i: (i, 0)),
          pl.BlockSpec((NC, BR, 32), lambda i: (0, i, 0)),
      ],
      out_shape=[
          jax.ShapeDtypeStruct((NACC, D), jnp.float32),
          jax.ShapeDtypeStruct((NC, NACC, 32), jnp.float32),
      ],
  )(n2, d2, asc, adst, gmax, hw2, b)


def _proj_body(h, w1, b1, w2, b2, w3, b3, o):
  r = jnp.maximum(_mm_t(h[...], w1[...]) + b1[...], 0.0)
  r = jnp.maximum(_mm_t(r, w2[...]) + b2[...], 0.0)
  o[...] = _mm_t(r, w3[...]) + b3[...]


def _proj(h, w1, b1, w2, b2, w3, b3):
  return pl.pallas_call(
      _proj_body,
      grid=(GRID,),
      in_specs=[
          pl.BlockSpec((BR, D), lambda i: (i, 0)),
          pl.BlockSpec((64, 64), lambda i: (0, 0)),
          pl.BlockSpec((1, 64), lambda i: (0, 0)),
          pl.BlockSpec((32, 64), lambda i: (0, 0)),
          pl.BlockSpec((1, 32), lambda i: (0, 0)),
          pl.BlockSpec((16, 32), lambda i: (0, 0)),
          pl.BlockSpec((1, 16), lambda i: (0, 0)),
      ],
      out_specs=pl.BlockSpec((BR, 16), lambda i: (i, 0)),
      out_shape=jax.ShapeDtypeStruct((NACC, 16), jnp.float32),
  )(h, w1, b1, w2, b2, w3, b3)


@jax.jit
def kernel(x, edge_index, sage_Wl, sage_Wr, sage_b, gat_W, gat_asrc, gat_adst,
           gat_b, pW1, pb1, pW2, pb2, pW3, pb3):
  src = edge_index[0]
  dst = edge_index[1]
  npad = EPAD - E
  src_r = jnp.concatenate([src, jnp.zeros((npad,), jnp.int32)]).reshape(
      EROWS, 128)
  dst_r = jnp.concatenate([dst, jnp.full((npad,), N, jnp.int32)]).reshape(
      EROWS, 128)

  cnt2 = _sc_count()(dst_r)

  xp = jnp.pad(x, ((0, NACC - N), (0, 0)))
  h = xp
  hh = jnp.stack([xp[:, :32], xp[:, 32:]])
  for i in range(4):
    s2 = _sc_rows(hh[0], hh[1], src_r, dst_r, jnp.zeros((EROWS, 128),
                                                        jnp.float32))
    h, hh = _sage_tc(s2, cnt2, h, sage_Wl[i], sage_Wr[i],
                     sage_b[i].reshape(1, D), i < 3)

  for i in range(3):
    hw2, asc, adst, gmax = _gat_pre(h, gat_W[i], gat_asrc[i].reshape(1, D),
                                    gat_adst[i].reshape(1, D))
    gmax16 = jnp.full((16,), 1.0, jnp.float32) * gmax[0, 0]
    ex_r, den2 = _sc_scalar()(asc.reshape(NACC), adst.reshape(NACC), src_r, dst_r,
                              gmax16)
    n2 = _sc_rows_w(hw2[0], hw2[1], src_r, dst_r, ex_r)
    h, hh = _gat_post(n2, den2, asc, adst, gmax, hw2,
                      gat_b[i].reshape(1, D), i < 2)

  out = _proj(h, pW1, pb1.reshape(1, 64), pW2, pb2.reshape(1, 32),
              pW3, pb3.reshape(1, 16))
  return out[:N]


# speedup vs baseline: 10.6638x; 1.0007x over previous
import functools

import jax
import jax.numpy as jnp
from jax import lax
from jax.experimental import pallas as pl
from jax.experimental.pallas import tpu as pltpu
from jax.experimental.pallas import tpu_sc as plsc

N = 50000
D = 64
E = 800000
NC = 2
NS = 16
LANES = 16
EPAD = 851968
EROWS = EPAD // 128
NACC = 51200
RPT = NACC // NS
CHUNK = 2048
ECH = 512
IR = ECH // 128
ROWS_T = EROWS // NS
ROWS_W = EROWS // (NS * NC)

@functools.lru_cache(maxsize=None)
def _mesh():
  return plsc.VectorSubcoreMesh(
      core_axis_name="c", subcore_axis_name="s", num_cores=NC,
      num_subcores=NS)


def _zero_flat(buf, nwords):
  z = jnp.zeros((16,), jnp.float32)
  def body(k, _):
    buf[pl.ds(k * 16, 16)] = z
    return 0
  lax.fori_loop(0, nwords // 16, body, 0)


def _zero_rows(buf, nrows):
  z = jnp.zeros((16,), jnp.float32)
  def body(e, _):
    buf[e, pl.ds(0, 16)] = z
    buf[e, pl.ds(16, 16)] = z
    return 0
  lax.fori_loop(0, nrows, body, 0)


def _count_body(dst_r, out, dbuf, ones, zflat, acc):
  cid = lax.axis_index("c")
  sid = lax.axis_index("s")
  wid = sid * NC + cid
  one = jnp.full((16,), 1.0, jnp.float32)
  for k in range(8):
    ones[0, pl.ds(k * 16, 16)] = one
  _zero_flat(zflat, CHUNK)
  base = sid * RPT
  pltpu.sync_copy(zflat, acc.at[pl.ds(base, CHUNK)])
  pltpu.sync_copy(zflat.at[pl.ds(0, RPT - CHUNK)],
                  acc.at[pl.ds(base + CHUNK, RPT - CHUNK)])
  plsc.subcore_barrier()
  def chunk(t, _):
    rowbase = wid * ROWS_W + t * 16
    pltpu.sync_copy(dst_r.at[pl.ds(rowbase, 16)], dbuf)
    for j in range(16):
      pltpu.sync_copy(ones.at[0], acc.at[dbuf.at[j]], add=True)
    return 0
  lax.fori_loop(0, ROWS_W // 16, chunk, 0)
  plsc.subcore_barrier()
  pltpu.sync_copy(acc.at[pl.ds(base, RPT)], out.at[cid, pl.ds(base, RPT)])


@functools.lru_cache(maxsize=None)
def _sc_count():
  return pl.kernel(
    _count_body,
    out_type=jax.ShapeDtypeStruct((NC, NACC), jnp.float32),
    mesh=_mesh(),
    compiler_params=pltpu.CompilerParams(use_tc_tiling_on_sc=False, needs_layout_passes=False),
    scratch_types=[
        pltpu.VMEM((16, 128), jnp.int32),
        pltpu.VMEM((1, 128), jnp.float32),
        pltpu.VMEM((CHUNK,), jnp.float32),
        pltpu.VMEM_SHARED((NACC,), jnp.float32),
    ],
  )


def _rows_body(h0, h1, src_r, dst_r, wt_r, out, sbuf, dbuf, wbuf, rows, acc,
               sem, *, weighted):
  cid = lax.axis_index("c")
  sid = lax.axis_index("s")
  _zero_rows(rows, ECH)
  base = sid * RPT
  for k in range(RPT // ECH):
    pltpu.sync_copy(rows, acc.at[pl.ds(base + k * ECH, ECH)])
  rem = RPT % ECH
  if rem:
    pltpu.sync_copy(rows.at[pl.ds(0, rem)],
                    acc.at[pl.ds(base + (RPT // ECH) * ECH, rem)])
  plsc.subcore_barrier()

  def main(table):
    def chunk(t, _):
      rowbase = sid * ROWS_T + t * IR
      pltpu.sync_copy(src_r.at[pl.ds(rowbase, IR)], sbuf)
      pltpu.sync_copy(dst_r.at[pl.ds(rowbase, IR)], dbuf)
      if weighted:
        pltpu.sync_copy(wt_r.at[pl.ds(rowbase, IR)], wbuf)
      cps = [pltpu.async_copy(table.at[sbuf.at[j]],
                              rows.at[pl.ds(j * 128, 128)], sem)
             for j in range(IR)]
      for cp in cps:
        cp.wait()
      if weighted:
        for j in range(IR):
          def scale(k, _):
            w16 = wbuf[j, pl.ds(k * 16, 16)]
            for l in range(16):
              e = j * 128 + k * 16 + l
              wb = jnp.take_along_axis(w16, jnp.full((16,), l, jnp.int32),
                                       axis=0)
              rows[e, pl.ds(0, 16)] = rows[e, pl.ds(0, 16)] * wb
              rows[e, pl.ds(16, 16)] = rows[e, pl.ds(16, 16)] * wb
            return 0
          lax.fori_loop(0, 8, scale, 0)
      for j in range(IR):
        pltpu.sync_copy(rows.at[pl.ds(j * 128, 128)],
                        acc.at[dbuf.at[j]], add=True)
      return 0
    lax.fori_loop(0, ROWS_T // IR, chunk, 0)

  @pl.when(cid == 0)
  def _():
    main(h0)

  @pl.when(cid == 1)
  def _():
    main(h1)

  plsc.subcore_barrier()
  pltpu.sync_copy(acc.at[pl.ds(base, RPT)], out.at[cid, pl.ds(base, RPT)])


@functools.lru_cache(maxsize=None)
def _make_rows_kernel(weighted):
  body = functools.partial(_rows_body, weighted=weighted)
  return pl.kernel(
      body,
      out_type=jax.ShapeDtypeStruct((NC, NACC, 32), jnp.float32),
      mesh=_mesh(),
      compiler_params=pltpu.CompilerParams(use_tc_tiling_on_sc=False, needs_layout_passes=False),
      scratch_types=[
          pltpu.VMEM((IR, 128), jnp.int32),
          pltpu.VMEM((IR, 128), jnp.int32),
          pltpu.VMEM((IR, 128), jnp.float32),
          pltpu.VMEM((ECH, 32), jnp.float32),
          pltpu.VMEM_SHARED((NACC, 32), jnp.float32),
          pltpu.SemaphoreType.DMA,
      ],
  )


def _sc_rows(*a):
  return _make_rows_kernel(False)(*a)


def _sc_rows_w(*a):
  return _make_rows_kernel(True)(*a)


def _scalar_body(asc_h, adst_h, src_r, dst_r, gmax_h, ex_out, den_out,
                 asct, adstt, sbuf, dbuf, exbuf, gv, zflat, acc):
  cid = lax.axis_index("c")
  sid = lax.axis_index("s")
  wid = sid * NC + cid
  pltpu.sync_copy(asc_h, asct)
  pltpu.sync_copy(adst_h, adstt)
  pltpu.sync_copy(gmax_h, gv)
  _zero_flat(zflat, CHUNK)
  base = sid * RPT
  pltpu.sync_copy(zflat, acc.at[pl.ds(base, CHUNK)])
  pltpu.sync_copy(zflat.at[pl.ds(0, RPT - CHUNK)],
                  acc.at[pl.ds(base + CHUNK, RPT - CHUNK)])
  plsc.subcore_barrier()
  g = gv[...]
  nmax = jnp.full((16,), N - 1, jnp.int32)
  def chunk(t, _):
    rowbase = wid * ROWS_W + t * 16
    pltpu.sync_copy(src_r.at[pl.ds(rowbase, 16)], sbuf)
    pltpu.sync_copy(dst_r.at[pl.ds(rowbase, 16)], dbuf)
    for j in range(16):
      def grp(k, _):
        s16 = sbuf[j, pl.ds(k * 16, 16)]
        d16 = jnp.minimum(dbuf[j, pl.ds(k * 16, 16)], nmax)
        a_s = plsc.load_gather(asct, [s16])
        a_d = plsc.load_gather(adstt, [d16])
        t1 = a_s + a_d
        lt = jnp.maximum(t1, 0.2 * t1)
        u = g + a_d
        lu = jnp.maximum(u, 0.2 * u)
        exbuf[j, pl.ds(k * 16, 16)] = jnp.exp(lt - lu)
        return 0
      lax.fori_loop(0, 8, grp, 0)
    pltpu.sync_copy(exbuf, ex_out.at[pl.ds(rowbase, 16)])
    for j in range(16):
      pltpu.sync_copy(exbuf.at[j], acc.at[dbuf.at[j]], add=True)
    return 0
  lax.fori_loop(0, ROWS_W // 16, chunk, 0)
  plsc.subcore_barrier()
  pltpu.sync_copy(acc.at[pl.ds(base, RPT)], den_out.at[cid, pl.ds(base, RPT)])


@functools.lru_cache(maxsize=None)
def _sc_scalar():
  return pl.kernel(
    _scalar_body,
    out_type=(jax.ShapeDtypeStruct((EROWS, 128), jnp.float32),
              jax.ShapeDtypeStruct((NC, NACC), jnp.float32)),
    mesh=_mesh(),
    compiler_params=pltpu.CompilerParams(use_tc_tiling_on_sc=False, needs_layout_passes=False),
    scratch_types=[
        pltpu.VMEM((NACC,), jnp.float32),
        pltpu.VMEM((NACC,), jnp.float32),
        pltpu.VMEM((16, 128), jnp.int32),
        pltpu.VMEM((16, 128), jnp.int32),
        pltpu.VMEM((16, 128), jnp.float32),
        pltpu.VMEM((16,), jnp.float32),
        pltpu.VMEM((CHUNK,), jnp.float32),
        pltpu.VMEM_SHARED((NACC,), jnp.float32),
    ],
  )


BR = 2048
GRID = NACC // BR


def _mm_t(a, w):
  return lax.dot_general(a, w, (((1,), (1,)), ((), ())),
                         preferred_element_type=jnp.float32)


def _sage_tc_body(s2, cnt2, h, wl, wr, b, o, oh, *, do_relu):
  s = jnp.concatenate([s2[0], s2[1]], axis=-1)
  cnt = jnp.clip(cnt2[0] + cnt2[1], 1.0, None)[:, None]
  res = _mm_t(s / cnt, wl[...]) + _mm_t(h[...], wr[...]) + b[...]
  if do_relu:
    res = jnp.maximum(res, 0.0)
  o[...] = res
  oh[0] = res[:, :32]
  oh[1] = res[:, 32:]


def _sage_tc(s2, cnt2, h, wl, wr, b, do_relu):
  body = functools.partial(_sage_tc_body, do_relu=do_relu)
  return pl.pallas_call(
      body,
      grid=(GRID,),
      in_specs=[
          pl.BlockSpec((NC, BR, 32), lambda i: (0, i, 0)),
          pl.BlockSpec((NC, BR), lambda i: (0, i)),
          pl.BlockSpec((BR, D), lambda i: (i, 0)),
          pl.BlockSpec((D, D), lambda i: (0, 0)),
          pl.BlockSpec((D, D), lambda i: (0, 0)),
          pl.BlockSpec((1, D), lambda i: (0, 0)),
      ],
      out_specs=[
          pl.BlockSpec((BR, D), lambda i: (i, 0)),
          pl.BlockSpec((NC, BR, 32), lambda i: (0, i, 0)),
      ],
      out_shape=[
          jax.ShapeDtypeStruct((NACC, D), jnp.float32),
          jax.ShapeDtypeStruct((NC, NACC, 32), jnp.float32),
      ],
  )(s2, cnt2, h, wl, wr, b)


def _gat_pre_body(h, w, a_s, a_d, ohw, oasc, oadst, ogmax):
  i = pl.program_id(0)
  hw = _mm_t(h[...], w[...])
  asc = _mm_t(hw, a_s[...])
  adst = _mm_t(hw, a_d[...])
  ohw[0] = hw[:, :32]
  ohw[1] = hw[:, 32:]
  oasc[...] = asc
  oadst[...] = adst
  bmax = jnp.max(asc)

  @pl.when(i == 0)
  def _():
    ogmax[0, 0] = bmax

  @pl.when(i > 0)
  def _():
    ogmax[0, 0] = jnp.maximum(ogmax[0, 0], bmax)


def _gat_pre(h, w, a_s2, a_d2):
  return pl.pallas_call(
      _gat_pre_body,
      grid=(GRID,),
      in_specs=[
          pl.BlockSpec((BR, D), lambda i: (i, 0)),
          pl.BlockSpec((D, D), lambda i: (0, 0)),
          pl.BlockSpec((1, D), lambda i: (0, 0)),
          pl.BlockSpec((1, D), lambda i: (0, 0)),
      ],
      out_specs=[
          pl.BlockSpec((NC, BR, 32), lambda i: (0, i, 0)),
          pl.BlockSpec((BR, 1), lambda i: (i, 0)),
          pl.BlockSpec((BR, 1), lambda i: (i, 0)),
          pl.BlockSpec((1, 1), lambda i: (0, 0), memory_space=pltpu.SMEM),
      ],
      out_shape=[
          jax.ShapeDtypeStruct((NC, NACC, 32), jnp.float32),
          jax.ShapeDtypeStruct((NACC, 1), jnp.float32),
          jax.ShapeDtypeStruct((NACC, 1), jnp.float32),
          jax.ShapeDtypeStruct((1, 1), jnp.float32),
      ],
  )(h, w, a_s2, a_d2)


def _gat_post_body(n2, d2, asc, adst, gmax, hw2, b, o, oh, *, do_relu):
  num = jnp.concatenate([n2[0], n2[1]], axis=-1)
  den = (d2[0] + d2[1])[:, None]
  hw = jnp.concatenate([hw2[0], hw2[1]], axis=-1)
  g = gmax[0, 0]
  t1 = asc[...] + adst[...]
  lt = jnp.maximum(t1, 0.2 * t1)
  u = g + adst[...]
  lu = jnp.maximum(u, 0.2 * u)
  exs = jnp.exp(lt - lu)
  res = (num + exs * hw) / (den + exs) + b[...]
  if do_relu:
    res = jnp.maximum(res, 0.0)
  o[...] = res
  oh[0] = res[:, :32]
  oh[1] = res[:, 32:]


def _gat_post(n2, d2, asc, adst, gmax, hw2, b, do_relu):
  body = functools.partial(_gat_post_body, do_relu=do_relu)
  return pl.pallas_call(
      body,
      grid=(GRID,),
      in_specs=[
          pl.BlockSpec((NC, BR, 32), lambda i: (0, i, 0)),
          pl.BlockSpec((NC, BR), lambda i: (0, i)),
          pl.BlockSpec((BR, 1), lambda i: (i, 0)),
          pl.BlockSpec((BR, 1), lambda i: (i, 0)),
          pl.BlockSpec((1, 1), lambda i: (0, 0), memory_space=pltpu.SMEM),
          pl.BlockSpec((NC, BR, 32), lambda i: (0, i, 0)),
          pl.BlockSpec((1, D), lambda i: (0, 0)),
      ],
      out_specs=[
          pl.BlockSpec((BR, D), lambda i: (i, 0)),
          pl.BlockSpec((NC, BR, 32), lambda i: (0, i, 0)),
      ],
      out_shape=[
          jax.ShapeDtypeStruct((NACC, D), jnp.float32),
          jax.ShapeDtypeStruct((NC, NACC, 32), jnp.float32),
      ],
  )(n2, d2, asc, adst, gmax, hw2, b)


def _proj_body(h, w1, b1, w2, b2, w3, b3, o):
  r = jnp.maximum(_mm_t(h[...], w1[...]) + b1[...], 0.0)
  r = jnp.maximum(_mm_t(r, w2[...]) + b2[...], 0.0)
  o[...] = _mm_t(r, w3[...]) + b3[...]


def _proj(h, w1, b1, w2, b2, w3, b3):
  return pl.pallas_call(
      _proj_body,
      grid=(GRID,),
      in_specs=[
          pl.BlockSpec((BR, D), lambda i: (i, 0)),
          pl.BlockSpec((64, 64), lambda i: (0, 0)),
          pl.BlockSpec((1, 64), lambda i: (0, 0)),
          pl.BlockSpec((32, 64), lambda i: (0, 0)),
          pl.BlockSpec((1, 32), lambda i: (0, 0)),
          pl.BlockSpec((16, 32), lambda i: (0, 0)),
          pl.BlockSpec((1, 16), lambda i: (0, 0)),
      ],
      out_specs=pl.BlockSpec((BR, 16), lambda i: (i, 0)),
      out_shape=jax.ShapeDtypeStruct((NACC, 16), jnp.float32),
  )(h, w1, b1, w2, b2, w3, b3)


@jax.jit
def kernel(x, edge_index, sage_Wl, sage_Wr, sage_b, gat_W, gat_asrc, gat_adst,
           gat_b, pW1, pb1, pW2, pb2, pW3, pb3):
  src = edge_index[0]
  dst = edge_index[1]
  npad = EPAD - E
  src_r = jnp.concatenate([src, jnp.zeros((npad,), jnp.int32)]).reshape(
      EROWS, 128)
  dump = N + (jnp.arange(npad, dtype=jnp.int32) % 1024)
  dst_r = jnp.concatenate([dst, dump]).reshape(EROWS, 128)

  cnt2 = _sc_count()(dst_r)

  xp = jnp.pad(x, ((0, NACC - N), (0, 0)))
  h = xp
  hh = jnp.stack([xp[:, :32], xp[:, 32:]])
  for i in range(4):
    s2 = _sc_rows(hh[0], hh[1], src_r, dst_r, jnp.zeros((EROWS, 128),
                                                        jnp.float32))
    h, hh = _sage_tc(s2, cnt2, h, sage_Wl[i], sage_Wr[i],
                     sage_b[i].reshape(1, D), i < 3)

  for i in range(3):
    hw2, asc, adst, gmax = _gat_pre(h, gat_W[i], gat_asrc[i].reshape(1, D),
                                    gat_adst[i].reshape(1, D))
    gmax16 = jnp.full((16,), 1.0, jnp.float32) * gmax[0, 0]
    ex_r, den2 = _sc_scalar()(asc.reshape(NACC), adst.reshape(NACC), src_r, dst_r,
                              gmax16)
    n2 = _sc_rows_w(hw2[0], hw2[1], src_r, dst_r, ex_r)
    h, hh = _gat_post(n2, den2, asc, adst, gmax, hw2,
                      gat_b[i].reshape(1, D), i < 2)

  out = _proj(h, pW1, pb1.reshape(1, 64), pW2, pb2.reshape(1, 32),
              pW3, pb3.reshape(1, 16))
  return out[:N]


# speedup vs baseline: 12.6318x; 1.1845x over previous
import functools

import jax
import jax.numpy as jnp
from jax import lax
from jax.experimental import pallas as pl
from jax.experimental.pallas import tpu as pltpu
from jax.experimental.pallas import tpu_sc as plsc

N = 50000
D = 64
E = 800000
NC = 2
NS = 16
LANES = 16
EPAD = 851968
EROWS = EPAD // 128
NACC = 51200
RPT = NACC // NS
CHUNK = 2048
ECH = 256
IR = ECH // 128
IDXB = 16
U = IDXB // IR
T_BLOCKS = (EROWS // NS) // IDXB
ROWS_T = EROWS // NS
ROWS_W = EROWS // (NS * NC)

@functools.lru_cache(maxsize=None)
def _mesh():
  return plsc.VectorSubcoreMesh(
      core_axis_name="c", subcore_axis_name="s", num_cores=NC,
      num_subcores=NS)


def _zero_flat(buf, nwords):
  z = jnp.zeros((16,), jnp.float32)
  def body(k, _):
    buf[pl.ds(k * 16, 16)] = z
    return 0
  lax.fori_loop(0, nwords // 16, body, 0)


def _zero_rows(buf, nrows):
  z = jnp.zeros((16,), jnp.float32)
  def body(e, _):
    buf[e, pl.ds(0, 16)] = z
    buf[e, pl.ds(16, 16)] = z
    return 0
  lax.fori_loop(0, nrows, body, 0)


def _count_body(dst_r, out, dbuf, ones, zflat, acc):
  cid = lax.axis_index("c")
  sid = lax.axis_index("s")
  wid = sid * NC + cid
  one = jnp.full((16,), 1.0, jnp.float32)
  for k in range(8):
    ones[0, pl.ds(k * 16, 16)] = one
  _zero_flat(zflat, CHUNK)
  base = sid * RPT
  pltpu.sync_copy(zflat, acc.at[pl.ds(base, CHUNK)])
  pltpu.sync_copy(zflat.at[pl.ds(0, RPT - CHUNK)],
                  acc.at[pl.ds(base + CHUNK, RPT - CHUNK)])
  plsc.subcore_barrier()
  def chunk(t, _):
    rowbase = wid * ROWS_W + t * 16
    pltpu.sync_copy(dst_r.at[pl.ds(rowbase, 16)], dbuf)
    for j in range(16):
      pltpu.sync_copy(ones.at[0], acc.at[dbuf.at[j]], add=True)
    return 0
  lax.fori_loop(0, ROWS_W // 16, chunk, 0)
  plsc.subcore_barrier()
  pltpu.sync_copy(acc.at[pl.ds(base, RPT)], out.at[cid, pl.ds(base, RPT)])


@functools.lru_cache(maxsize=None)
def _sc_count():
  return pl.kernel(
    _count_body,
    out_type=jax.ShapeDtypeStruct((NC, NACC), jnp.float32),
    mesh=_mesh(),
    compiler_params=pltpu.CompilerParams(use_tc_tiling_on_sc=False, needs_layout_passes=False),
    scratch_types=[
        pltpu.VMEM((16, 128), jnp.int32),
        pltpu.VMEM((1, 128), jnp.float32),
        pltpu.VMEM((CHUNK,), jnp.float32),
        pltpu.VMEM_SHARED((NACC,), jnp.float32),
    ],
  )


def _rows_body(h0, h1, src_r, dst_r, wt_r, out, sbuf, dbuf, wbuf, rows, acc,
               sem, sem2, *, weighted):
  cid = lax.axis_index("c")
  sid = lax.axis_index("s")
  z = jnp.zeros((16,), jnp.float32)
  for s in range(2):
    def zbody(e, _):
      rows[s, e, pl.ds(0, 16)] = z
      rows[s, e, pl.ds(16, 16)] = z
      return 0
    lax.fori_loop(0, ECH, zbody, 0)
  base = sid * RPT
  for k in range(RPT // ECH):
    pltpu.sync_copy(rows.at[0], acc.at[pl.ds(base + k * ECH, ECH)])
  rem = RPT % ECH
  if rem:
    pltpu.sync_copy(rows.at[0, pl.ds(0, rem)],
                    acc.at[pl.ds(base + (RPT // ECH) * ECH, rem)])
  plsc.subcore_barrier()

  def main(table):
    def block(tb, _):
      rowbase = sid * ROWS_T + tb * IDXB
      pltpu.sync_copy(src_r.at[pl.ds(rowbase, IDXB)], sbuf)
      pltpu.sync_copy(dst_r.at[pl.ds(rowbase, IDXB)], dbuf)
      if weighted:
        pltpu.sync_copy(wt_r.at[pl.ds(rowbase, IDXB)], wbuf)

      def fire_gather(u):
        s = u % 2
        return [pltpu.async_copy(table.at[sbuf.at[u * IR + j]],
                                 rows.at[s, pl.ds(j * 128, 128)], sem)
                for j in range(IR)]

      def fire_scatter(u):
        s = u % 2
        return [pltpu.async_copy(rows.at[s, pl.ds(j * 128, 128)],
                                 acc.at[dbuf.at[u * IR + j]], sem2, add=True)
                for j in range(IR)]

      g = {0: fire_gather(0)}
      sc = {}
      for u in range(U):
        if u >= 1:
          for cp in sc[u - 1]:
            cp.wait()
        if u + 1 < U:
          g[u + 1] = fire_gather(u + 1)
        for cp in g[u]:
          cp.wait()
        if weighted:
          s = u % 2
          for j in range(IR):
            row_j = u * IR + j
            def scale(k, _, s=s, j=j, row_j=row_j):
              w16 = wbuf[row_j, pl.ds(k * 16, 16)]
              for l in range(16):
                e = j * 128 + k * 16 + l
                wb = jnp.take_along_axis(w16, jnp.full((16,), l, jnp.int32),
                                         axis=0)
                rows[s, e, pl.ds(0, 16)] = rows[s, e, pl.ds(0, 16)] * wb
                rows[s, e, pl.ds(16, 16)] = rows[s, e, pl.ds(16, 16)] * wb
              return 0
            lax.fori_loop(0, 8, scale, 0)
        sc[u] = fire_scatter(u)
      for cp in sc[U - 1]:
        cp.wait()
      return 0
    lax.fori_loop(0, T_BLOCKS, block, 0)

  @pl.when(cid == 0)
  def _():
    main(h0)

  @pl.when(cid == 1)
  def _():
    main(h1)

  plsc.subcore_barrier()
  pltpu.sync_copy(acc.at[pl.ds(base, RPT)], out.at[cid, pl.ds(base, RPT)])


@functools.lru_cache(maxsize=None)
def _make_rows_kernel(weighted):
  body = functools.partial(_rows_body, weighted=weighted)
  return pl.kernel(
      body,
      out_type=jax.ShapeDtypeStruct((NC, NACC, 32), jnp.float32),
      mesh=_mesh(),
      compiler_params=pltpu.CompilerParams(use_tc_tiling_on_sc=False, needs_layout_passes=False),
      scratch_types=[
          pltpu.VMEM((IDXB, 128), jnp.int32),
          pltpu.VMEM((IDXB, 128), jnp.int32),
          pltpu.VMEM((IDXB, 128), jnp.float32),
          pltpu.VMEM((2, ECH, 32), jnp.float32),
          pltpu.VMEM_SHARED((NACC, 32), jnp.float32),
          pltpu.SemaphoreType.DMA,
          pltpu.SemaphoreType.DMA,
      ],
  )


def _sc_rows(*a):
  return _make_rows_kernel(False)(*a)


def _sc_rows_w(*a):
  return _make_rows_kernel(True)(*a)


def _scalar_body(asc_h, adst_h, src_r, dst_r, gmax_h, ex_out, den_out,
                 asct, adstt, sbuf, dbuf, exbuf, gv, zflat, acc):
  cid = lax.axis_index("c")
  sid = lax.axis_index("s")
  wid = sid * NC + cid
  pltpu.sync_copy(asc_h, asct)
  pltpu.sync_copy(adst_h, adstt)
  pltpu.sync_copy(gmax_h, gv)
  _zero_flat(zflat, CHUNK)
  base = sid * RPT
  pltpu.sync_copy(zflat, acc.at[pl.ds(base, CHUNK)])
  pltpu.sync_copy(zflat.at[pl.ds(0, RPT - CHUNK)],
                  acc.at[pl.ds(base + CHUNK, RPT - CHUNK)])
  plsc.subcore_barrier()
  g = gv[...]
  nmax = jnp.full((16,), N - 1, jnp.int32)
  def chunk(t, _):
    rowbase = wid * ROWS_W + t * 16
    pltpu.sync_copy(src_r.at[pl.ds(rowbase, 16)], sbuf)
    pltpu.sync_copy(dst_r.at[pl.ds(rowbase, 16)], dbuf)
    for j in range(16):
      def grp(k, _):
        s16 = sbuf[j, pl.ds(k * 16, 16)]
        d16 = jnp.minimum(dbuf[j, pl.ds(k * 16, 16)], nmax)
        a_s = plsc.load_gather(asct, [s16])
        a_d = plsc.load_gather(adstt, [d16])
        t1 = a_s + a_d
        lt = jnp.maximum(t1, 0.2 * t1)
        u = g + a_d
        lu = jnp.maximum(u, 0.2 * u)
        exbuf[j, pl.ds(k * 16, 16)] = jnp.exp(lt - lu)
        return 0
      lax.fori_loop(0, 8, grp, 0)
    pltpu.sync_copy(exbuf, ex_out.at[pl.ds(rowbase, 16)])
    for j in range(16):
      pltpu.sync_copy(exbuf.at[j], acc.at[dbuf.at[j]], add=True)
    return 0
  lax.fori_loop(0, ROWS_W // 16, chunk, 0)
  plsc.subcore_barrier()
  pltpu.sync_copy(acc.at[pl.ds(base, RPT)], den_out.at[cid, pl.ds(base, RPT)])


@functools.lru_cache(maxsize=None)
def _sc_scalar():
  return pl.kernel(
    _scalar_body,
    out_type=(jax.ShapeDtypeStruct((EROWS, 128), jnp.float32),
              jax.ShapeDtypeStruct((NC, NACC), jnp.float32)),
    mesh=_mesh(),
    compiler_params=pltpu.CompilerParams(use_tc_tiling_on_sc=False, needs_layout_passes=False),
    scratch_types=[
        pltpu.VMEM((NACC,), jnp.float32),
        pltpu.VMEM((NACC,), jnp.float32),
        pltpu.VMEM((16, 128), jnp.int32),
        pltpu.VMEM((16, 128), jnp.int32),
        pltpu.VMEM((16, 128), jnp.float32),
        pltpu.VMEM((16,), jnp.float32),
        pltpu.VMEM((CHUNK,), jnp.float32),
        pltpu.VMEM_SHARED((NACC,), jnp.float32),
    ],
  )


BR = 2048
GRID = NACC // BR


def _mm_t(a, w):
  return lax.dot_general(a, w, (((1,), (1,)), ((), ())),
                         preferred_element_type=jnp.float32)


def _sage_tc_body(s2, cnt2, h, wl, wr, b, o, oh, *, do_relu):
  s = jnp.concatenate([s2[0], s2[1]], axis=-1)
  cnt = jnp.clip(cnt2[0] + cnt2[1], 1.0, None)[:, None]
  res = _mm_t(s / cnt, wl[...]) + _mm_t(h[...], wr[...]) + b[...]
  if do_relu:
    res = jnp.maximum(res, 0.0)
  o[...] = res
  oh[0] = res[:, :32]
  oh[1] = res[:, 32:]


def _sage_tc(s2, cnt2, h, wl, wr, b, do_relu):
  body = functools.partial(_sage_tc_body, do_relu=do_relu)
  return pl.pallas_call(
      body,
      grid=(GRID,),
      in_specs=[
          pl.BlockSpec((NC, BR, 32), lambda i: (0, i, 0)),
          pl.BlockSpec((NC, BR), lambda i: (0, i)),
          pl.BlockSpec((BR, D), lambda i: (i, 0)),
          pl.BlockSpec((D, D), lambda i: (0, 0)),
          pl.BlockSpec((D, D), lambda i: (0, 0)),
          pl.BlockSpec((1, D), lambda i: (0, 0)),
      ],
      out_specs=[
          pl.BlockSpec((BR, D), lambda i: (i, 0)),
          pl.BlockSpec((NC, BR, 32), lambda i: (0, i, 0)),
      ],
      out_shape=[
          jax.ShapeDtypeStruct((NACC, D), jnp.float32),
          jax.ShapeDtypeStruct((NC, NACC, 32), jnp.float32),
      ],
  )(s2, cnt2, h, wl, wr, b)


def _gat_pre_body(h, w, a_s, a_d, ohw, oasc, oadst, ogmax):
  i = pl.program_id(0)
  hw = _mm_t(h[...], w[...])
  asc = _mm_t(hw, a_s[...])
  adst = _mm_t(hw, a_d[...])
  ohw[0] = hw[:, :32]
  ohw[1] = hw[:, 32:]
  oasc[...] = asc
  oadst[...] = adst
  bmax = jnp.max(asc)

  @pl.when(i == 0)
  def _():
    ogmax[0, 0] = bmax

  @pl.when(i > 0)
  def _():
    ogmax[0, 0] = jnp.maximum(ogmax[0, 0], bmax)


def _gat_pre(h, w, a_s2, a_d2):
  return pl.pallas_call(
      _gat_pre_body,
      grid=(GRID,),
      in_specs=[
          pl.BlockSpec((BR, D), lambda i: (i, 0)),
          pl.BlockSpec((D, D), lambda i: (0, 0)),
          pl.BlockSpec((1, D), lambda i: (0, 0)),
          pl.BlockSpec((1, D), lambda i: (0, 0)),
      ],
      out_specs=[
          pl.BlockSpec((NC, BR, 32), lambda i: (0, i, 0)),
          pl.BlockSpec((BR, 1), lambda i: (i, 0)),
          pl.BlockSpec((BR, 1), lambda i: (i, 0)),
          pl.BlockSpec((1, 1), lambda i: (0, 0), memory_space=pltpu.SMEM),
      ],
      out_shape=[
          jax.ShapeDtypeStruct((NC, NACC, 32), jnp.float32),
          jax.ShapeDtypeStruct((NACC, 1), jnp.float32),
          jax.ShapeDtypeStruct((NACC, 1), jnp.float32),
          jax.ShapeDtypeStruct((1, 1), jnp.float32),
      ],
  )(h, w, a_s2, a_d2)


def _gat_post_body(n2, d2, asc, adst, gmax, hw2, b, o, oh, *, do_relu):
  num = jnp.concatenate([n2[0], n2[1]], axis=-1)
  den = (d2[0] + d2[1])[:, None]
  hw = jnp.concatenate([hw2[0], hw2[1]], axis=-1)
  g = gmax[0, 0]
  t1 = asc[...] + adst[...]
  lt = jnp.maximum(t1, 0.2 * t1)
  u = g + adst[...]
  lu = jnp.maximum(u, 0.2 * u)
  exs = jnp.exp(lt - lu)
  res = (num + exs * hw) / (den + exs) + b[...]
  if do_relu:
    res = jnp.maximum(res, 0.0)
  o[...] = res
  oh[0] = res[:, :32]
  oh[1] = res[:, 32:]


def _gat_post(n2, d2, asc, adst, gmax, hw2, b, do_relu):
  body = functools.partial(_gat_post_body, do_relu=do_relu)
  return pl.pallas_call(
      body,
      grid=(GRID,),
      in_specs=[
          pl.BlockSpec((NC, BR, 32), lambda i: (0, i, 0)),
          pl.BlockSpec((NC, BR), lambda i: (0, i)),
          pl.BlockSpec((BR, 1), lambda i: (i, 0)),
          pl.BlockSpec((BR, 1), lambda i: (i, 0)),
          pl.BlockSpec((1, 1), lambda i: (0, 0), memory_space=pltpu.SMEM),
          pl.BlockSpec((NC, BR, 32), lambda i: (0, i, 0)),
          pl.BlockSpec((1, D), lambda i: (0, 0)),
      ],
      out_specs=[
          pl.BlockSpec((BR, D), lambda i: (i, 0)),
          pl.BlockSpec((NC, BR, 32), lambda i: (0, i, 0)),
      ],
      out_shape=[
          jax.ShapeDtypeStruct((NACC, D), jnp.float32),
          jax.ShapeDtypeStruct((NC, NACC, 32), jnp.float32),
      ],
  )(n2, d2, asc, adst, gmax, hw2, b)


def _proj_body(h, w1, b1, w2, b2, w3, b3, o):
  r = jnp.maximum(_mm_t(h[...], w1[...]) + b1[...], 0.0)
  r = jnp.maximum(_mm_t(r, w2[...]) + b2[...], 0.0)
  o[...] = _mm_t(r, w3[...]) + b3[...]


def _proj(h, w1, b1, w2, b2, w3, b3):
  return pl.pallas_call(
      _proj_body,
      grid=(GRID,),
      in_specs=[
          pl.BlockSpec((BR, D), lambda i: (i, 0)),
          pl.BlockSpec((64, 64), lambda i: (0, 0)),
          pl.BlockSpec((1, 64), lambda i: (0, 0)),
          pl.BlockSpec((32, 64), lambda i: (0, 0)),
          pl.BlockSpec((1, 32), lambda i: (0, 0)),
          pl.BlockSpec((16, 32), lambda i: (0, 0)),
          pl.BlockSpec((1, 16), lambda i: (0, 0)),
      ],
      out_specs=pl.BlockSpec((BR, 16), lambda i: (i, 0)),
      out_shape=jax.ShapeDtypeStruct((NACC, 16), jnp.float32),
  )(h, w1, b1, w2, b2, w3, b3)


@jax.jit
def kernel(x, edge_index, sage_Wl, sage_Wr, sage_b, gat_W, gat_asrc, gat_adst,
           gat_b, pW1, pb1, pW2, pb2, pW3, pb3):
  src = edge_index[0]
  dst = edge_index[1]
  npad = EPAD - E
  src_r = jnp.concatenate([src, jnp.zeros((npad,), jnp.int32)]).reshape(
      EROWS, 128)
  dump = N + (jnp.arange(npad, dtype=jnp.int32) % 1024)
  dst_r = jnp.concatenate([dst, dump]).reshape(EROWS, 128)

  cnt2 = _sc_count()(dst_r)

  xp = jnp.pad(x, ((0, NACC - N), (0, 0)))
  h = xp
  hh = jnp.stack([xp[:, :32], xp[:, 32:]])
  for i in range(4):
    s2 = _sc_rows(hh[0], hh[1], src_r, dst_r, jnp.zeros((EROWS, 128),
                                                        jnp.float32))
    h, hh = _sage_tc(s2, cnt2, h, sage_Wl[i], sage_Wr[i],
                     sage_b[i].reshape(1, D), i < 3)

  for i in range(3):
    hw2, asc, adst, gmax = _gat_pre(h, gat_W[i], gat_asrc[i].reshape(1, D),
                                    gat_adst[i].reshape(1, D))
    gmax16 = jnp.full((16,), 1.0, jnp.float32) * gmax[0, 0]
    ex_r, den2 = _sc_scalar()(asc.reshape(NACC), adst.reshape(NACC), src_r, dst_r,
                              gmax16)
    n2 = _sc_rows_w(hw2[0], hw2[1], src_r, dst_r, ex_r)
    h, hh = _gat_post(n2, den2, asc, adst, gmax, hw2,
                      gat_b[i].reshape(1, D), i < 2)

  out = _proj(h, pW1, pb1.reshape(1, 64), pW2, pb2.reshape(1, 32),
              pW3, pb3.reshape(1, 16))
  return out[:N]


# speedup vs baseline: 14.5724x; 1.1536x over previous
import functools

import jax
import jax.numpy as jnp
from jax import lax
from jax.experimental import pallas as pl
from jax.experimental.pallas import tpu as pltpu
from jax.experimental.pallas import tpu_sc as plsc

N = 50000
D = 64
E = 800000
NC = 2
NS = 16
LANES = 16
EPAD = 851968
EROWS = EPAD // 128
NACC = 51200
RPT = NACC // NS
CHUNK = 2048
ECH = 256
IR = ECH // 128
IDXB = 16
U = IDXB // IR
T_BLOCKS = (EROWS // NS) // IDXB
ROWS_T = EROWS // NS
ROWS_W = EROWS // (NS * NC)

@functools.lru_cache(maxsize=None)
def _mesh():
  return plsc.VectorSubcoreMesh(
      core_axis_name="c", subcore_axis_name="s", num_cores=NC,
      num_subcores=NS)


def _zero_flat(buf, nwords):
  z = jnp.zeros((16,), jnp.float32)
  def body(k, _):
    buf[pl.ds(k * 16, 16)] = z
    return 0
  lax.fori_loop(0, nwords // 16, body, 0)


def _zero_rows(buf, nrows):
  z = jnp.zeros((16,), jnp.float32)
  def body(e, _):
    buf[e, pl.ds(0, 16)] = z
    buf[e, pl.ds(16, 16)] = z
    return 0
  lax.fori_loop(0, nrows, body, 0)


def _count_body(dst_r, out, dbuf, ones, zflat, acc):
  cid = lax.axis_index("c")
  sid = lax.axis_index("s")
  wid = sid * NC + cid
  one = jnp.full((16,), 1.0, jnp.float32)
  for k in range(8):
    ones[0, pl.ds(k * 16, 16)] = one
  _zero_flat(zflat, CHUNK)
  base = sid * RPT
  pltpu.sync_copy(zflat, acc.at[pl.ds(base, CHUNK)])
  pltpu.sync_copy(zflat.at[pl.ds(0, RPT - CHUNK)],
                  acc.at[pl.ds(base + CHUNK, RPT - CHUNK)])
  plsc.subcore_barrier()
  def chunk(t, _):
    rowbase = wid * ROWS_W + t * 16
    pltpu.sync_copy(dst_r.at[pl.ds(rowbase, 16)], dbuf)
    for j in range(16):
      pltpu.sync_copy(ones.at[0], acc.at[dbuf.at[j]], add=True)
    return 0
  lax.fori_loop(0, ROWS_W // 16, chunk, 0)
  plsc.subcore_barrier()
  pltpu.sync_copy(acc.at[pl.ds(base, RPT)], out.at[cid, pl.ds(base, RPT)])


@functools.lru_cache(maxsize=None)
def _sc_count():
  return pl.kernel(
    _count_body,
    out_type=jax.ShapeDtypeStruct((NC, NACC), jnp.float32),
    mesh=_mesh(),
    compiler_params=pltpu.CompilerParams(use_tc_tiling_on_sc=False, needs_layout_passes=False),
    scratch_types=[
        pltpu.VMEM((16, 128), jnp.int32),
        pltpu.VMEM((1, 128), jnp.float32),
        pltpu.VMEM((CHUNK,), jnp.float32),
        pltpu.VMEM_SHARED((NACC,), jnp.float32),
    ],
  )


def _rows_body(h0, h1, src_r, dst_r, wt_r, *rest, weighted, with_count):
  if with_count:
    (out, cnt_out, sbuf, dbuf, wbuf, rows, acc, sem, sem2, ones, zflat,
     cacc) = rest
  else:
    out, sbuf, dbuf, wbuf, rows, acc, sem, sem2 = rest
  cid = lax.axis_index("c")
  sid = lax.axis_index("s")
  z = jnp.zeros((16,), jnp.float32)
  for s in range(2):
    def zbody(e, _):
      rows[s, e, pl.ds(0, 16)] = z
      rows[s, e, pl.ds(16, 16)] = z
      return 0
    lax.fori_loop(0, ECH, zbody, 0)
  base = sid * RPT
  for k in range(RPT // ECH):
    pltpu.sync_copy(rows.at[0], acc.at[pl.ds(base + k * ECH, ECH)])
  rem = RPT % ECH
  if rem:
    pltpu.sync_copy(rows.at[0, pl.ds(0, rem)],
                    acc.at[pl.ds(base + (RPT // ECH) * ECH, rem)])
  if with_count:
    one = jnp.full((16,), 1.0, jnp.float32)
    for k in range(8):
      ones[0, pl.ds(k * 16, 16)] = one
    _zero_flat(zflat, 256)
    for k in range(RPT // 256):
      pltpu.sync_copy(zflat, cacc.at[pl.ds(base + k * 256, 256)])
    crem = RPT % 256
    if crem:
      pltpu.sync_copy(zflat.at[pl.ds(0, crem)],
                      cacc.at[pl.ds(base + (RPT // 256) * 256, crem)])
  plsc.subcore_barrier()

  def main(table):
    def block(tb, _):
      rowbase = sid * ROWS_T + tb * IDXB
      pltpu.sync_copy(src_r.at[pl.ds(rowbase, IDXB)], sbuf)
      pltpu.sync_copy(dst_r.at[pl.ds(rowbase, IDXB)], dbuf)
      if weighted:
        pltpu.sync_copy(wt_r.at[pl.ds(rowbase, IDXB)], wbuf)

      def fire_gather(u):
        s = u % 2
        return [pltpu.async_copy(table.at[sbuf.at[u * IR + j]],
                                 rows.at[s, pl.ds(j * 128, 128)], sem)
                for j in range(IR)]

      def fire_scatter(u):
        s = u % 2
        cps = [pltpu.async_copy(rows.at[s, pl.ds(j * 128, 128)],
                                acc.at[dbuf.at[u * IR + j]], sem2, add=True)
               for j in range(IR)]
        if with_count:
          cps += [pltpu.async_copy(ones.at[0], cacc.at[dbuf.at[u * IR + j]],
                                   sem2, add=True) for j in range(IR)]
        return cps

      g = {0: fire_gather(0)}
      sc = {}
      for u in range(U):
        if u >= 1:
          for cp in sc[u - 1]:
            cp.wait()
        if u + 1 < U:
          g[u + 1] = fire_gather(u + 1)
        for cp in g[u]:
          cp.wait()
        if weighted:
          s = u % 2
          for j in range(IR):
            row_j = u * IR + j
            def scale(k, _, s=s, j=j, row_j=row_j):
              w16 = wbuf[row_j, pl.ds(k * 16, 16)]
              for l in range(16):
                e = j * 128 + k * 16 + l
                wb = jnp.take_along_axis(w16, jnp.full((16,), l, jnp.int32),
                                         axis=0)
                rows[s, e, pl.ds(0, 16)] = rows[s, e, pl.ds(0, 16)] * wb
                rows[s, e, pl.ds(16, 16)] = rows[s, e, pl.ds(16, 16)] * wb
              return 0
            lax.fori_loop(0, 8, scale, 0)
        sc[u] = fire_scatter(u)
      for cp in sc[U - 1]:
        cp.wait()
      return 0
    lax.fori_loop(0, T_BLOCKS, block, 0)

  @pl.when(cid == 0)
  def _():
    main(h0)

  @pl.when(cid == 1)
  def _():
    main(h1)

  plsc.subcore_barrier()
  pltpu.sync_copy(acc.at[pl.ds(base, RPT)], out.at[cid, pl.ds(base, RPT)])
  if with_count:
    pltpu.sync_copy(cacc.at[pl.ds(base, RPT)],
                    cnt_out.at[cid, pl.ds(base, RPT)])


@functools.lru_cache(maxsize=None)
def _make_rows_kernel(weighted, with_count=False):
  body = functools.partial(_rows_body, weighted=weighted,
                           with_count=with_count)
  out_type = jax.ShapeDtypeStruct((NC, NACC, 32), jnp.float32)
  if with_count:
    out_type = (out_type, jax.ShapeDtypeStruct((NC, NACC), jnp.float32))
  scratch = [
      pltpu.VMEM((IDXB, 128), jnp.int32),
      pltpu.VMEM((IDXB, 128), jnp.int32),
      pltpu.VMEM((IDXB, 128), jnp.float32),
      pltpu.VMEM((2, ECH, 32), jnp.float32),
      pltpu.VMEM_SHARED((NACC, 32), jnp.float32),
      pltpu.SemaphoreType.DMA,
      pltpu.SemaphoreType.DMA,
  ]
  if with_count:
    scratch += [
        pltpu.VMEM((1, 128), jnp.float32),
        pltpu.VMEM((256,), jnp.float32),
        pltpu.VMEM_SHARED((NACC,), jnp.float32),
    ]
  return pl.kernel(
      body,
      out_type=out_type,
      mesh=_mesh(),
      compiler_params=pltpu.CompilerParams(use_tc_tiling_on_sc=False, needs_layout_passes=False),
      scratch_types=scratch,
  )


def _sc_rows(*a):
  return _make_rows_kernel(False)(*a)


def _sc_rows_cnt(*a):
  return _make_rows_kernel(False, True)(*a)


def _sc_rows_w(*a):
  return _make_rows_kernel(True)(*a)


def _scalar_body(asc_h, adst_h, src_r, dst_r, gmax_h, ex_out, den_out,
                 asct, adstt, sbuf, dbuf, exbuf, gv, zflat, acc):
  cid = lax.axis_index("c")
  sid = lax.axis_index("s")
  wid = sid * NC + cid
  pltpu.sync_copy(asc_h, asct)
  pltpu.sync_copy(adst_h, adstt)
  pltpu.sync_copy(gmax_h, gv)
  _zero_flat(zflat, CHUNK)
  base = sid * RPT
  pltpu.sync_copy(zflat, acc.at[pl.ds(base, CHUNK)])
  pltpu.sync_copy(zflat.at[pl.ds(0, RPT - CHUNK)],
                  acc.at[pl.ds(base + CHUNK, RPT - CHUNK)])
  plsc.subcore_barrier()
  g = gv[...]
  nmax = jnp.full((16,), N - 1, jnp.int32)
  def chunk(t, _):
    rowbase = wid * ROWS_W + t * 16
    pltpu.sync_copy(src_r.at[pl.ds(rowbase, 16)], sbuf)
    pltpu.sync_copy(dst_r.at[pl.ds(rowbase, 16)], dbuf)
    for j in range(16):
      def grp(k, _):
        s16 = sbuf[j, pl.ds(k * 16, 16)]
        d16 = jnp.minimum(dbuf[j, pl.ds(k * 16, 16)], nmax)
        a_s = plsc.load_gather(asct, [s16])
        a_d = plsc.load_gather(adstt, [d16])
        t1 = a_s + a_d
        lt = jnp.maximum(t1, 0.2 * t1)
        u = g + a_d
        lu = jnp.maximum(u, 0.2 * u)
        exbuf[j, pl.ds(k * 16, 16)] = jnp.exp(lt - lu)
        return 0
      lax.fori_loop(0, 8, grp, 0)
    pltpu.sync_copy(exbuf, ex_out.at[pl.ds(rowbase, 16)])
    for j in range(16):
      pltpu.sync_copy(exbuf.at[j], acc.at[dbuf.at[j]], add=True)
    return 0
  lax.fori_loop(0, ROWS_W // 16, chunk, 0)
  plsc.subcore_barrier()
  pltpu.sync_copy(acc.at[pl.ds(base, RPT)], den_out.at[cid, pl.ds(base, RPT)])


@functools.lru_cache(maxsize=None)
def _sc_scalar():
  return pl.kernel(
    _scalar_body,
    out_type=(jax.ShapeDtypeStruct((EROWS, 128), jnp.float32),
              jax.ShapeDtypeStruct((NC, NACC), jnp.float32)),
    mesh=_mesh(),
    compiler_params=pltpu.CompilerParams(use_tc_tiling_on_sc=False, needs_layout_passes=False),
    scratch_types=[
        pltpu.VMEM((NACC,), jnp.float32),
        pltpu.VMEM((NACC,), jnp.float32),
        pltpu.VMEM((16, 128), jnp.int32),
        pltpu.VMEM((16, 128), jnp.int32),
        pltpu.VMEM((16, 128), jnp.float32),
        pltpu.VMEM((16,), jnp.float32),
        pltpu.VMEM((CHUNK,), jnp.float32),
        pltpu.VMEM_SHARED((NACC,), jnp.float32),
    ],
  )


BR = 2048
GRID = NACC // BR


def _mm_t(a, w):
  return lax.dot_general(a, w, (((1,), (1,)), ((), ())),
                         preferred_element_type=jnp.float32)


def _sage_tc_body(s2, cnt2, h, wl, wr, b, o, oh, *, do_relu):
  s = jnp.concatenate([s2[0], s2[1]], axis=-1)
  cnt = jnp.clip(0.5 * (cnt2[0] + cnt2[1]), 1.0, None)[:, None]
  res = _mm_t(s / cnt, wl[...]) + _mm_t(h[...], wr[...]) + b[...]
  if do_relu:
    res = jnp.maximum(res, 0.0)
  o[...] = res
  oh[0] = res[:, :32]
  oh[1] = res[:, 32:]


def _sage_tc(s2, cnt2, h, wl, wr, b, do_relu):
  body = functools.partial(_sage_tc_body, do_relu=do_relu)
  return pl.pallas_call(
      body,
      grid=(GRID,),
      in_specs=[
          pl.BlockSpec((NC, BR, 32), lambda i: (0, i, 0)),
          pl.BlockSpec((NC, BR), lambda i: (0, i)),
          pl.BlockSpec((BR, D), lambda i: (i, 0)),
          pl.BlockSpec((D, D), lambda i: (0, 0)),
          pl.BlockSpec((D, D), lambda i: (0, 0)),
          pl.BlockSpec((1, D), lambda i: (0, 0)),
      ],
      out_specs=[
          pl.BlockSpec((BR, D), lambda i: (i, 0)),
          pl.BlockSpec((NC, BR, 32), lambda i: (0, i, 0)),
      ],
      out_shape=[
          jax.ShapeDtypeStruct((NACC, D), jnp.float32),
          jax.ShapeDtypeStruct((NC, NACC, 32), jnp.float32),
      ],
  )(s2, cnt2, h, wl, wr, b)


def _gatpre_emit(res, w, a_s, a_d, ohw, oasc, oadst, ogmax):
  i = pl.program_id(0)
  hw = _mm_t(res, w[...])
  asc = _mm_t(hw, a_s[...])
  adst = _mm_t(hw, a_d[...])
  ohw[0] = hw[:, :32]
  ohw[1] = hw[:, 32:]
  oasc[...] = asc
  oadst[...] = adst
  bmax = jnp.max(asc)

  @pl.when(i == 0)
  def _():
    ogmax[0, 0] = bmax

  @pl.when(i > 0)
  def _():
    ogmax[0, 0] = jnp.maximum(ogmax[0, 0], bmax)


_GATPRE_OUT_SPECS = [
    pl.BlockSpec((NC, BR, 32), lambda i: (0, i, 0)),
    pl.BlockSpec((BR, 1), lambda i: (i, 0)),
    pl.BlockSpec((BR, 1), lambda i: (i, 0)),
    pl.BlockSpec((1, 1), lambda i: (0, 0), memory_space=pltpu.SMEM),
]
_GATPRE_OUT_SHAPE = [
    jax.ShapeDtypeStruct((NC, NACC, 32), jnp.float32),
    jax.ShapeDtypeStruct((NACC, 1), jnp.float32),
    jax.ShapeDtypeStruct((NACC, 1), jnp.float32),
    jax.ShapeDtypeStruct((1, 1), jnp.float32),
]


def _sage_gatpre_body(s2, cnt2, h, wl, wr, b, w, a_s, a_d,
                      ohw, oasc, oadst, ogmax):
  s = jnp.concatenate([s2[0], s2[1]], axis=-1)
  cnt = jnp.clip(0.5 * (cnt2[0] + cnt2[1]), 1.0, None)[:, None]
  res = _mm_t(s / cnt, wl[...]) + _mm_t(h[...], wr[...]) + b[...]
  _gatpre_emit(res, w, a_s, a_d, ohw, oasc, oadst, ogmax)


def _sage_gatpre(s2, cnt2, h, wl, wr, b, w, a_s2, a_d2):
  return pl.pallas_call(
      _sage_gatpre_body,
      grid=(GRID,),
      in_specs=[
          pl.BlockSpec((NC, BR, 32), lambda i: (0, i, 0)),
          pl.BlockSpec((NC, BR), lambda i: (0, i)),
          pl.BlockSpec((BR, D), lambda i: (i, 0)),
          pl.BlockSpec((D, D), lambda i: (0, 0)),
          pl.BlockSpec((D, D), lambda i: (0, 0)),
          pl.BlockSpec((1, D), lambda i: (0, 0)),
          pl.BlockSpec((D, D), lambda i: (0, 0)),
          pl.BlockSpec((1, D), lambda i: (0, 0)),
          pl.BlockSpec((1, D), lambda i: (0, 0)),
      ],
      out_specs=_GATPRE_OUT_SPECS,
      out_shape=_GATPRE_OUT_SHAPE,
  )(s2, cnt2, h, wl, wr, b, w, a_s2, a_d2)


def _gat_res(n2, d2, asc, adst, gmax, hw2, b, do_relu):
  num = jnp.concatenate([n2[0], n2[1]], axis=-1)
  den = (d2[0] + d2[1])[:, None]
  hw = jnp.concatenate([hw2[0], hw2[1]], axis=-1)
  g = gmax[0, 0]
  t1 = asc[...] + adst[...]
  lt = jnp.maximum(t1, 0.2 * t1)
  u = g + adst[...]
  lu = jnp.maximum(u, 0.2 * u)
  exs = jnp.exp(lt - lu)
  res = (num + exs * hw) / (den + exs) + b[...]
  if do_relu:
    res = jnp.maximum(res, 0.0)
  return res


_GATPOST_IN_SPECS = [
    pl.BlockSpec((NC, BR, 32), lambda i: (0, i, 0)),
    pl.BlockSpec((NC, BR), lambda i: (0, i)),
    pl.BlockSpec((BR, 1), lambda i: (i, 0)),
    pl.BlockSpec((BR, 1), lambda i: (i, 0)),
    pl.BlockSpec((1, 1), lambda i: (0, 0), memory_space=pltpu.SMEM),
    pl.BlockSpec((NC, BR, 32), lambda i: (0, i, 0)),
    pl.BlockSpec((1, D), lambda i: (0, 0)),
]


def _gat_post_pre_body(n2, d2, asc, adst, gmax, hw2, b, w, a_s, a_d,
                       ohw, oasc, oadst, ogmax):
  res = _gat_res(n2, d2, asc, adst, gmax, hw2, b, True)
  _gatpre_emit(res, w, a_s, a_d, ohw, oasc, oadst, ogmax)


def _gat_post_pre(n2, d2, asc, adst, gmax, hw2, b, w, a_s2, a_d2):
  return pl.pallas_call(
      _gat_post_pre_body,
      grid=(GRID,),
      in_specs=_GATPOST_IN_SPECS + [
          pl.BlockSpec((D, D), lambda i: (0, 0)),
          pl.BlockSpec((1, D), lambda i: (0, 0)),
          pl.BlockSpec((1, D), lambda i: (0, 0)),
      ],
      out_specs=_GATPRE_OUT_SPECS,
      out_shape=_GATPRE_OUT_SHAPE,
  )(n2, d2, asc, adst, gmax, hw2, b, w, a_s2, a_d2)


def _gat_post_proj_body(n2, d2, asc, adst, gmax, hw2, b, w1, b1, w2, b2,
                        w3, b3, o):
  res = _gat_res(n2, d2, asc, adst, gmax, hw2, b, False)
  r = jnp.maximum(_mm_t(res, w1[...]) + b1[...], 0.0)
  r = jnp.maximum(_mm_t(r, w2[...]) + b2[...], 0.0)
  o[...] = _mm_t(r, w3[...]) + b3[...]


def _gat_post_proj(n2, d2, asc, adst, gmax, hw2, b, w1, b1, w2, b2, w3, b3):
  return pl.pallas_call(
      _gat_post_proj_body,
      grid=(GRID,),
      in_specs=_GATPOST_IN_SPECS + [
          pl.BlockSpec((64, 64), lambda i: (0, 0)),
          pl.BlockSpec((1, 64), lambda i: (0, 0)),
          pl.BlockSpec((32, 64), lambda i: (0, 0)),
          pl.BlockSpec((1, 32), lambda i: (0, 0)),
          pl.BlockSpec((16, 32), lambda i: (0, 0)),
          pl.BlockSpec((1, 16), lambda i: (0, 0)),
      ],
      out_specs=pl.BlockSpec((BR, 16), lambda i: (i, 0)),
      out_shape=jax.ShapeDtypeStruct((NACC, 16), jnp.float32),
  )(n2, d2, asc, adst, gmax, hw2, b, w1, b1, w2, b2, w3, b3)


@jax.jit
def kernel(x, edge_index, sage_Wl, sage_Wr, sage_b, gat_W, gat_asrc, gat_adst,
           gat_b, pW1, pb1, pW2, pb2, pW3, pb3):
  src = edge_index[0]
  dst = edge_index[1]
  npad = EPAD - E
  src_r = jnp.concatenate([src, jnp.zeros((npad,), jnp.int32)]).reshape(
      EROWS, 128)
  dump = N + (jnp.arange(npad, dtype=jnp.int32) % 1024)
  dst_r = jnp.concatenate([dst, dump]).reshape(EROWS, 128)

  xp = jnp.pad(x, ((0, NACC - N), (0, 0)))
  h = xp
  hh = jnp.stack([xp[:, :32], xp[:, 32:]])
  wz = jnp.zeros((EROWS, 128), jnp.float32)

  s2, cnt2 = _sc_rows_cnt(hh[0], hh[1], src_r, dst_r, wz)
  for i in range(3):
    if i > 0:
      s2 = _sc_rows(hh[0], hh[1], src_r, dst_r, wz)
    h, hh = _sage_tc(s2, cnt2, h, sage_Wl[i], sage_Wr[i],
                     sage_b[i].reshape(1, D), True)

  s2 = _sc_rows(hh[0], hh[1], src_r, dst_r, wz)
  hw2, asc, adst, gmax = _sage_gatpre(
      s2, cnt2, h, sage_Wl[3], sage_Wr[3], sage_b[3].reshape(1, D),
      gat_W[0], gat_asrc[0].reshape(1, D), gat_adst[0].reshape(1, D))

  for i in range(3):
    gmax16 = jnp.full((16,), 1.0, jnp.float32) * gmax[0, 0]
    ex_r, den2 = _sc_scalar()(asc.reshape(NACC), adst.reshape(NACC), src_r,
                              dst_r, gmax16)
    n2 = _sc_rows_w(hw2[0], hw2[1], src_r, dst_r, ex_r)
    if i < 2:
      hw2, asc, adst, gmax = _gat_post_pre(
          n2, den2, asc, adst, gmax, hw2, gat_b[i].reshape(1, D),
          gat_W[i + 1], gat_asrc[i + 1].reshape(1, D),
          gat_adst[i + 1].reshape(1, D))
    else:
      out = _gat_post_proj(
          n2, den2, asc, adst, gmax, hw2, gat_b[i].reshape(1, D),
          pW1, pb1.reshape(1, 64), pW2, pb2.reshape(1, 32),
          pW3, pb3.reshape(1, 16))
  return out[:N]


# speedup vs baseline: 14.5757x; 1.0002x over previous
import functools

import jax
import jax.numpy as jnp
from jax import lax
from jax.experimental import pallas as pl
from jax.experimental.pallas import tpu as pltpu
from jax.experimental.pallas import tpu_sc as plsc

N = 50000
D = 64
E = 800000
NC = 2
NS = 16
LANES = 16
EPAD = 851968
EROWS = EPAD // 128
NACC = 51200
RPT = NACC // NS
CHUNK = 2048
ECH = 256
IR = ECH // 128
IDXB = 16
U = IDXB // IR
T_BLOCKS = (EROWS // NS) // IDXB
ROWS_T = EROWS // NS
ROWS_W = EROWS // (NS * NC)

@functools.lru_cache(maxsize=None)
def _mesh():
  return plsc.VectorSubcoreMesh(
      core_axis_name="c", subcore_axis_name="s", num_cores=NC,
      num_subcores=NS)


def _zero_flat(buf, nwords):
  z = jnp.zeros((16,), jnp.float32)
  def body(k, _):
    buf[pl.ds(k * 16, 16)] = z
    return 0
  lax.fori_loop(0, nwords // 16, body, 0)


def _rows_body(h0, h1, src_r, dst_r, wt_r, *rest, weighted, with_count):
  if with_count:
    (out, cnt_out, sbuf, dbuf, wbuf, rows, acc, sem, sem2, ones, zflat,
     cacc) = rest
  else:
    out, sbuf, dbuf, wbuf, rows, acc, sem, sem2 = rest
  cid = lax.axis_index("c")
  sid = lax.axis_index("s")
  z = jnp.zeros((16,), jnp.float32)
  for s in range(2):
    def zbody(e, _):
      rows[s, e, pl.ds(0, 16)] = z
      rows[s, e, pl.ds(16, 16)] = z
      return 0
    lax.fori_loop(0, ECH, zbody, 0)
  base = sid * RPT
  for k in range(RPT // ECH):
    pltpu.sync_copy(rows.at[0], acc.at[pl.ds(base + k * ECH, ECH)])
  rem = RPT % ECH
  if rem:
    pltpu.sync_copy(rows.at[0, pl.ds(0, rem)],
                    acc.at[pl.ds(base + (RPT // ECH) * ECH, rem)])
  if with_count:
    one = jnp.full((16,), 1.0, jnp.float32)
    for k in range(8):
      ones[0, pl.ds(k * 16, 16)] = one
    _zero_flat(zflat, 256)
    for k in range(RPT // 256):
      pltpu.sync_copy(zflat, cacc.at[pl.ds(base + k * 256, 256)])
    crem = RPT % 256
    if crem:
      pltpu.sync_copy(zflat.at[pl.ds(0, crem)],
                      cacc.at[pl.ds(base + (RPT // 256) * 256, crem)])
  plsc.subcore_barrier()

  def main(table):
    def block(tb, _):
      rowbase = sid * ROWS_T + tb * IDXB
      pltpu.sync_copy(src_r.at[pl.ds(rowbase, IDXB)], sbuf)
      pltpu.sync_copy(dst_r.at[pl.ds(rowbase, IDXB)], dbuf)
      if weighted:
        pltpu.sync_copy(wt_r.at[pl.ds(rowbase, IDXB)], wbuf)

      def fire_gather(u):
        s = u % 2
        return [pltpu.async_copy(table.at[sbuf.at[u * IR + j]],
                                 rows.at[s, pl.ds(j * 128, 128)], sem)
                for j in range(IR)]

      def fire_scatter(u):
        s = u % 2
        cps = [pltpu.async_copy(rows.at[s, pl.ds(j * 128, 128)],
                                acc.at[dbuf.at[u * IR + j]], sem2, add=True)
               for j in range(IR)]
        if with_count:
          cps += [pltpu.async_copy(ones.at[0], cacc.at[dbuf.at[u * IR + j]],
                                   sem2, add=True) for j in range(IR)]
        return cps

      g = {0: fire_gather(0)}
      sc = {}
      for u in range(U):
        if u >= 1:
          for cp in sc[u - 1]:
            cp.wait()
        if u + 1 < U:
          g[u + 1] = fire_gather(u + 1)
        for cp in g[u]:
          cp.wait()
        if weighted:
          s = u % 2
          for j in range(IR):
            row_j = u * IR + j
            def scale(k, _, s=s, j=j, row_j=row_j):
              w16 = wbuf[row_j, pl.ds(k * 16, 16)]
              for l in range(16):
                e = j * 128 + k * 16 + l
                wb = jnp.take_along_axis(w16, jnp.full((16,), l, jnp.int32),
                                         axis=0)
                rows[s, e, pl.ds(0, 16)] = rows[s, e, pl.ds(0, 16)] * wb
                rows[s, e, pl.ds(16, 16)] = rows[s, e, pl.ds(16, 16)] * wb
              return 0
            lax.fori_loop(0, 8, scale, 0)
        sc[u] = fire_scatter(u)
      for cp in sc[U - 1]:
        cp.wait()
      return 0
    lax.fori_loop(0, T_BLOCKS, block, 0)

  @pl.when(cid == 0)
  def _():
    main(h0)

  @pl.when(cid == 1)
  def _():
    main(h1)

  plsc.subcore_barrier()
  pltpu.sync_copy(acc.at[pl.ds(base, RPT)], out.at[cid, pl.ds(base, RPT)])
  if with_count:
    pltpu.sync_copy(cacc.at[pl.ds(base, RPT)],
                    cnt_out.at[cid, pl.ds(base, RPT)])


@functools.lru_cache(maxsize=None)
def _make_rows_kernel(weighted, with_count=False):
  body = functools.partial(_rows_body, weighted=weighted,
                           with_count=with_count)
  out_type = jax.ShapeDtypeStruct((NC, NACC, 32), jnp.float32)
  if with_count:
    out_type = (out_type, jax.ShapeDtypeStruct((NC, NACC), jnp.float32))
  scratch = [
      pltpu.VMEM((IDXB, 128), jnp.int32),
      pltpu.VMEM((IDXB, 128), jnp.int32),
      pltpu.VMEM((IDXB, 128), jnp.float32),
      pltpu.VMEM((2, ECH, 32), jnp.float32),
      pltpu.VMEM_SHARED((NACC, 32), jnp.float32),
      pltpu.SemaphoreType.DMA,
      pltpu.SemaphoreType.DMA,
  ]
  if with_count:
    scratch += [
        pltpu.VMEM((1, 128), jnp.float32),
        pltpu.VMEM((256,), jnp.float32),
        pltpu.VMEM_SHARED((NACC,), jnp.float32),
    ]
  return pl.kernel(
      body,
      out_type=out_type,
      mesh=_mesh(),
      compiler_params=pltpu.CompilerParams(use_tc_tiling_on_sc=False, needs_layout_passes=False),
      scratch_types=scratch,
  )


def _sc_rows(*a):
  return _make_rows_kernel(False)(*a)


def _sc_rows_cnt(*a):
  return _make_rows_kernel(False, True)(*a)


def _sc_rows_w(*a):
  return _make_rows_kernel(True)(*a)


def _scalar_body(asc_h, adst_h, src_r, dst_r, gmax_h, ex_out, den_out,
                 asct, adstt, sbuf, dbuf, exbuf, gv, zflat, acc):
  cid = lax.axis_index("c")
  sid = lax.axis_index("s")
  wid = sid * NC + cid
  pltpu.sync_copy(asc_h, asct)
  pltpu.sync_copy(adst_h, adstt)
  pltpu.sync_copy(gmax_h, gv)
  _zero_flat(zflat, CHUNK)
  base = sid * RPT
  pltpu.sync_copy(zflat, acc.at[pl.ds(base, CHUNK)])
  pltpu.sync_copy(zflat.at[pl.ds(0, RPT - CHUNK)],
                  acc.at[pl.ds(base + CHUNK, RPT - CHUNK)])
  plsc.subcore_barrier()
  g = gv[...]
  nmax = jnp.full((16,), N - 1, jnp.int32)
  def chunk(t, _):
    rowbase = wid * ROWS_W + t * 16
    pltpu.sync_copy(src_r.at[pl.ds(rowbase, 16)], sbuf)
    pltpu.sync_copy(dst_r.at[pl.ds(rowbase, 16)], dbuf)
    for j in range(16):
      def grp(k, _):
        s16 = sbuf[j, pl.ds(k * 16, 16)]
        d16 = jnp.minimum(dbuf[j, pl.ds(k * 16, 16)], nmax)
        a_s = plsc.load_gather(asct, [s16])
        a_d = plsc.load_gather(adstt, [d16])
        t1 = a_s + a_d
        lt = jnp.maximum(t1, 0.2 * t1)
        u = g + a_d
        lu = jnp.maximum(u, 0.2 * u)
        exbuf[j, pl.ds(k * 16, 16)] = jnp.exp(lt - lu)
        return 0
      lax.fori_loop(0, 8, grp, 0)
    pltpu.sync_copy(exbuf, ex_out.at[pl.ds(rowbase, 16)])
    for j in range(16):
      pltpu.sync_copy(exbuf.at[j], acc.at[dbuf.at[j]], add=True)
    return 0
  lax.fori_loop(0, ROWS_W // 16, chunk, 0)
  plsc.subcore_barrier()
  pltpu.sync_copy(acc.at[pl.ds(base, RPT)], den_out.at[cid, pl.ds(base, RPT)])


@functools.lru_cache(maxsize=None)
def _sc_scalar():
  return pl.kernel(
    _scalar_body,
    out_type=(jax.ShapeDtypeStruct((EROWS, 128), jnp.float32),
              jax.ShapeDtypeStruct((NC, NACC), jnp.float32)),
    mesh=_mesh(),
    compiler_params=pltpu.CompilerParams(use_tc_tiling_on_sc=False, needs_layout_passes=False),
    scratch_types=[
        pltpu.VMEM((NACC,), jnp.float32),
        pltpu.VMEM((NACC,), jnp.float32),
        pltpu.VMEM((16, 128), jnp.int32),
        pltpu.VMEM((16, 128), jnp.int32),
        pltpu.VMEM((16, 128), jnp.float32),
        pltpu.VMEM((16,), jnp.float32),
        pltpu.VMEM((CHUNK,), jnp.float32),
        pltpu.VMEM_SHARED((NACC,), jnp.float32),
    ],
  )


BR = 2048
GRID = NACC // BR


def _mm_t(a, w):
  return lax.dot_general(a, w, (((1,), (1,)), ((), ())),
                         preferred_element_type=jnp.float32)


def _sage_tc_body(s2, cnt2, h, wl, wr, b, o, oh, *, do_relu):
  s = jnp.concatenate([s2[0], s2[1]], axis=-1)
  cnt = jnp.clip(0.5 * (cnt2[0] + cnt2[1]), 1.0, None)[:, None]
  res = _mm_t(s / cnt, wl[...]) + _mm_t(h[...], wr[...]) + b[...]
  if do_relu:
    res = jnp.maximum(res, 0.0)
  o[...] = res
  oh[0] = res[:, :32]
  oh[1] = res[:, 32:]


def _sage_tc(s2, cnt2, h, wl, wr, b, do_relu):
  body = functools.partial(_sage_tc_body, do_relu=do_relu)
  return pl.pallas_call(
      body,
      grid=(GRID,),
      in_specs=[
          pl.BlockSpec((NC, BR, 32), lambda i: (0, i, 0)),
          pl.BlockSpec((NC, BR), lambda i: (0, i)),
          pl.BlockSpec((BR, D), lambda i: (i, 0)),
          pl.BlockSpec((D, D), lambda i: (0, 0)),
          pl.BlockSpec((D, D), lambda i: (0, 0)),
          pl.BlockSpec((1, D), lambda i: (0, 0)),
      ],
      out_specs=[
          pl.BlockSpec((BR, D), lambda i: (i, 0)),
          pl.BlockSpec((NC, BR, 32), lambda i: (0, i, 0)),
      ],
      out_shape=[
          jax.ShapeDtypeStruct((NACC, D), jnp.float32),
          jax.ShapeDtypeStruct((NC, NACC, 32), jnp.float32),
      ],
  )(s2, cnt2, h, wl, wr, b)


def _gatpre_emit(res, w, a_s, a_d, ohw, oasc, oadst, ogmax):
  i = pl.program_id(0)
  hw = _mm_t(res, w[...])
  asc = _mm_t(hw, a_s[...])
  adst = _mm_t(hw, a_d[...])
  ohw[0] = hw[:, :32]
  ohw[1] = hw[:, 32:]
  oasc[...] = asc
  oadst[...] = adst
  bmax = jnp.max(asc)

  @pl.when(i == 0)
  def _():
    ogmax[0, 0] = bmax

  @pl.when(i > 0)
  def _():
    ogmax[0, 0] = jnp.maximum(ogmax[0, 0], bmax)


_GATPRE_OUT_SPECS = [
    pl.BlockSpec((NC, BR, 32), lambda i: (0, i, 0)),
    pl.BlockSpec((BR, 1), lambda i: (i, 0)),
    pl.BlockSpec((BR, 1), lambda i: (i, 0)),
    pl.BlockSpec((1, 1), lambda i: (0, 0), memory_space=pltpu.SMEM),
]
_GATPRE_OUT_SHAPE = [
    jax.ShapeDtypeStruct((NC, NACC, 32), jnp.float32),
    jax.ShapeDtypeStruct((NACC, 1), jnp.float32),
    jax.ShapeDtypeStruct((NACC, 1), jnp.float32),
    jax.ShapeDtypeStruct((1, 1), jnp.float32),
]


def _sage_gatpre_body(s2, cnt2, h, wl, wr, b, w, a_s, a_d,
                      ohw, oasc, oadst, ogmax):
  s = jnp.concatenate([s2[0], s2[1]], axis=-1)
  cnt = jnp.clip(0.5 * (cnt2[0] + cnt2[1]), 1.0, None)[:, None]
  res = _mm_t(s / cnt, wl[...]) + _mm_t(h[...], wr[...]) + b[...]
  _gatpre_emit(res, w, a_s, a_d, ohw, oasc, oadst, ogmax)


def _sage_gatpre(s2, cnt2, h, wl, wr, b, w, a_s2, a_d2):
  return pl.pallas_call(
      _sage_gatpre_body,
      grid=(GRID,),
      in_specs=[
          pl.BlockSpec((NC, BR, 32), lambda i: (0, i, 0)),
          pl.BlockSpec((NC, BR), lambda i: (0, i)),
          pl.BlockSpec((BR, D), lambda i: (i, 0)),
          pl.BlockSpec((D, D), lambda i: (0, 0)),
          pl.BlockSpec((D, D), lambda i: (0, 0)),
          pl.BlockSpec((1, D), lambda i: (0, 0)),
          pl.BlockSpec((D, D), lambda i: (0, 0)),
          pl.BlockSpec((1, D), lambda i: (0, 0)),
          pl.BlockSpec((1, D), lambda i: (0, 0)),
      ],
      out_specs=_GATPRE_OUT_SPECS,
      out_shape=_GATPRE_OUT_SHAPE,
  )(s2, cnt2, h, wl, wr, b, w, a_s2, a_d2)


def _gat_res(n2, d2, asc, adst, gmax, hw2, b, do_relu):
  num = jnp.concatenate([n2[0], n2[1]], axis=-1)
  den = (d2[0] + d2[1])[:, None]
  hw = jnp.concatenate([hw2[0], hw2[1]], axis=-1)
  g = gmax[0, 0]
  t1 = asc[...] + adst[...]
  lt = jnp.maximum(t1, 0.2 * t1)
  u = g + adst[...]
  lu = jnp.maximum(u, 0.2 * u)
  exs = jnp.exp(lt - lu)
  res = (num + exs * hw) / (den + exs) + b[...]
  if do_relu:
    res = jnp.maximum(res, 0.0)
  return res


_GATPOST_IN_SPECS = [
    pl.BlockSpec((NC, BR, 32), lambda i: (0, i, 0)),
    pl.BlockSpec((NC, BR), lambda i: (0, i)),
    pl.BlockSpec((BR, 1), lambda i: (i, 0)),
    pl.BlockSpec((BR, 1), lambda i: (i, 0)),
    pl.BlockSpec((1, 1), lambda i: (0, 0), memory_space=pltpu.SMEM),
    pl.BlockSpec((NC, BR, 32), lambda i: (0, i, 0)),
    pl.BlockSpec((1, D), lambda i: (0, 0)),
]


def _gat_post_pre_body(n2, d2, asc, adst, gmax, hw2, b, w, a_s, a_d,
                       ohw, oasc, oadst, ogmax):
  res = _gat_res(n2, d2, asc, adst, gmax, hw2, b, True)
  _gatpre_emit(res, w, a_s, a_d, ohw, oasc, oadst, ogmax)


def _gat_post_pre(n2, d2, asc, adst, gmax, hw2, b, w, a_s2, a_d2):
  return pl.pallas_call(
      _gat_post_pre_body,
      grid=(GRID,),
      in_specs=_GATPOST_IN_SPECS + [
          pl.BlockSpec((D, D), lambda i: (0, 0)),
          pl.BlockSpec((1, D), lambda i: (0, 0)),
          pl.BlockSpec((1, D), lambda i: (0, 0)),
      ],
      out_specs=_GATPRE_OUT_SPECS,
      out_shape=_GATPRE_OUT_SHAPE,
  )(n2, d2, asc, adst, gmax, hw2, b, w, a_s2, a_d2)


def _gat_post_proj_body(n2, d2, asc, adst, gmax, hw2, b, w1, b1, w2, b2,
                        w3, b3, o):
  res = _gat_res(n2, d2, asc, adst, gmax, hw2, b, False)
  r = jnp.maximum(_mm_t(res, w1[...]) + b1[...], 0.0)
  r = jnp.maximum(_mm_t(r, w2[...]) + b2[...], 0.0)
  o[...] = _mm_t(r, w3[...]) + b3[...]


def _gat_post_proj(n2, d2, asc, adst, gmax, hw2, b, w1, b1, w2, b2, w3, b3):
  return pl.pallas_call(
      _gat_post_proj_body,
      grid=(GRID,),
      in_specs=_GATPOST_IN_SPECS + [
          pl.BlockSpec((64, 64), lambda i: (0, 0)),
          pl.BlockSpec((1, 64), lambda i: (0, 0)),
          pl.BlockSpec((32, 64), lambda i: (0, 0)),
          pl.BlockSpec((1, 32), lambda i: (0, 0)),
          pl.BlockSpec((16, 32), lambda i: (0, 0)),
          pl.BlockSpec((1, 16), lambda i: (0, 0)),
      ],
      out_specs=pl.BlockSpec((BR, 16), lambda i: (i, 0)),
      out_shape=jax.ShapeDtypeStruct((NACC, 16), jnp.float32),
  )(n2, d2, asc, adst, gmax, hw2, b, w1, b1, w2, b2, w3, b3)


@jax.jit
def kernel(x, edge_index, sage_Wl, sage_Wr, sage_b, gat_W, gat_asrc, gat_adst,
           gat_b, pW1, pb1, pW2, pb2, pW3, pb3):
  src = edge_index[0]
  dst = edge_index[1]
  npad = EPAD - E
  src_r = jnp.concatenate([src, jnp.zeros((npad,), jnp.int32)]).reshape(
      EROWS, 128)
  dump = N + (jnp.arange(npad, dtype=jnp.int32) % 1024)
  dst_r = jnp.concatenate([dst, dump]).reshape(EROWS, 128)

  xp = jnp.pad(x, ((0, NACC - N), (0, 0)))
  h = xp
  hh = jnp.stack([xp[:, :32], xp[:, 32:]])
  wz = jnp.zeros((EROWS, 128), jnp.float32)

  s2, cnt2 = _sc_rows_cnt(hh[0], hh[1], src_r, dst_r, wz)
  for i in range(3):
    if i > 0:
      s2 = _sc_rows(hh[0], hh[1], src_r, dst_r, wz)
    h, hh = _sage_tc(s2, cnt2, h, sage_Wl[i], sage_Wr[i],
                     sage_b[i].reshape(1, D), True)

  s2 = _sc_rows(hh[0], hh[1], src_r, dst_r, wz)
  hw2, asc, adst, gmax = _sage_gatpre(
      s2, cnt2, h, sage_Wl[3], sage_Wr[3], sage_b[3].reshape(1, D),
      gat_W[0], gat_asrc[0].reshape(1, D), gat_adst[0].reshape(1, D))

  for i in range(3):
    gmax16 = jnp.full((16,), 1.0, jnp.float32) * gmax[0, 0]
    ex_r, den2 = _sc_scalar()(asc.reshape(NACC), adst.reshape(NACC), src_r,
                              dst_r, gmax16)
    n2 = _sc_rows_w(hw2[0], hw2[1], src_r, dst_r, ex_r)
    if i < 2:
      hw2, asc, adst, gmax = _gat_post_pre(
          n2, den2, asc, adst, gmax, hw2, gat_b[i].reshape(1, D),
          gat_W[i + 1], gat_asrc[i + 1].reshape(1, D),
          gat_adst[i + 1].reshape(1, D))
    else:
      out = _gat_post_proj(
          n2, den2, asc, adst, gmax, hw2, gat_b[i].reshape(1, D),
          pW1, pb1.reshape(1, 64), pW2, pb2.reshape(1, 32),
          pW3, pb3.reshape(1, 16))
  return out[:N]
